# Initial kernel scaffold; baseline (speedup 1.0000x reference)
import jax
import jax.numpy as jnp
from jax.experimental import pallas as pl


def kernel(edge_list, edge_type, batch_inputs, params):
    raise NotImplementedError("write your pallas kernel here")


# speedup vs baseline: 1.0001x; 1.0001x over previous
import jax
import jax.numpy as jnp
from jax.experimental import pallas as pl

N_NODES = 10000
N_REL = 64
IN_DIM = 128
REL_DIM = 64
NHID = 16
NHEADS = 2
D1 = NHID * NHEADS
N_EDGES = 160000
BATCH = 8192
ALPHA = 0.2
EPS = 1e-5
FILT_W = 9
OUT_CH = 32
CONV_W = D1 - FILT_W + 1


def _leaky(x):
    return jnp.where(x > 0, x, ALPHA * x)


def _l2norm(x):
    return x / jnp.maximum(jnp.linalg.norm(x, axis=1, keepdims=True), 1e-12)


def _sp_att(h, edge, edge_embed, a, a2, concat):
    eh = jnp.concatenate([h[edge[0]], h[edge[1]], edge_embed], axis=1)
    em = eh @ a.T
    powers = -_leaky(em @ a2.T)
    ee = jnp.exp(powers[:, 0])
    rowsum = jax.ops.segment_sum(ee, edge[0], num_segments=h.shape[0])
    hp = jax.ops.segment_sum(ee[:, None] * em, edge[0], num_segments=h.shape[0])
    hp = hp / (rowsum[:, None] + 1e-12)
    return jax.nn.elu(hp) if concat else hp


def _hyper(ent, edge, p):
    ue = ent @ p['W_E']
    head = ue[edge[0]]
    tail = ue[edge[1]]
    mu0 = head.mean()
    v0 = head.var()
    x = (head - mu0) / jnp.sqrt(v0 + EPS) * p['bn0_g'][0] + p['bn0_b'][0]
    k = (tail @ p['fc1_w'] + p['fc1_b']).reshape(-1, OUT_CH, FILT_W)
    idx = jnp.arange(CONV_W)[:, None] + jnp.arange(FILT_W)[None, :]
    xw = x[:, idx]
    conv = jnp.einsum('ejw,eow->eoj', xw, k)
    mu1 = conv.mean(axis=(0, 2), keepdims=True)
    v1 = conv.var(axis=(0, 2), keepdims=True)
    conv = (conv - mu1) / jnp.sqrt(v1 + EPS) * p['bn1_g'][None, :, None] + p['bn1_b'][None, :, None]
    flat = conv.reshape(conv.shape[0], -1)
    out = flat @ p['fc_w'] + p['fc_b']
    mu2 = out.mean(axis=0)
    v2 = out.var(axis=0)
    out = (out - mu2) / jnp.sqrt(v2 + EPS) * p['bn2_g'] + p['bn2_b']
    return jax.nn.relu(out)


def kernel(edge_list, edge_type, batch_inputs, params):
    p = params
    ent = _l2norm(p['entity_embeddings'])
    rel = p['relation_embeddings']
    edge_embed = rel[edge_type]
    heads = [_sp_att(ent, edge_list, edge_embed, p['att_a'][i], p['att_a2'][i], True) for i in range(NHEADS)]
    x = jnp.concatenate(heads, axis=1)
    rel1 = (rel @ p['W_1'])[edge_type]
    out_entity_1 = jax.nn.elu(_sp_att(x, edge_list, rel1, p['out_a'], p['out_a2'], False))
    hyper_out = _hyper(ent, edge_list, p)
    rel_sum = jax.ops.segment_sum(hyper_out, edge_type, num_segments=N_REL)
    counts = jnp.maximum(jnp.bincount(edge_type, length=N_REL), 1).astype(jnp.float32)
    out_relation = rel_sum / counts[:, None]
    mask = jnp.zeros((N_NODES,), jnp.float32).at[batch_inputs[:, 2]].set(1.0)
    out_entity = ent @ p['W_entities'] + mask[:, None] * out_entity_1
    return (_l2norm(out_entity), _l2norm(out_relation))


# speedup vs baseline: 3.0151x; 3.0149x over previous
import functools

import jax
import jax.numpy as jnp
import numpy as np
from jax import lax
from jax.experimental import pallas as pl
from jax.experimental.pallas import tpu as pltpu
from jax.experimental.pallas import tpu_sc as plsc

N_NODES = 10000
N_REL = 64
IN_DIM = 128
REL_DIM = 64
NHID = 16
NHEADS = 2
D1 = NHID * NHEADS
N_EDGES = 160000
BATCH = 8192
ALPHA = 0.2
EPS = 1e-5
FILT_W = 9
OUT_CH = 32
CONV_W = D1 - FILT_W + 1
NCR = OUT_CH * CONV_W

EB = 1280
NBE = N_EDGES // EB
PW = 128
TW = 128
NPAD = 10240
CH = 128
F32 = jnp.float32

_INTERPRET = False


def _leaky(x):
    return jnp.where(x > 0, x, ALPHA * x)


def _elu(x):
    return jnp.where(x > 0, x, jnp.exp(jnp.minimum(x, 0.0)) - 1.0)


def _pc(body, out_shape, **kw):
    return pl.pallas_call(body, out_shape=out_shape, interpret=_INTERPRET, **kw)


def _k1_body(e_ref, asrc_ref, adst_ref, we_ref, went_ref, rel_ref, arel_ref,
             w1_ref, oar_ref, tsrc_ref, tdst_ref, ew_ref, re_ref,
             r1p_ref):
    e = e_ref[...]
    nrm = jnp.sqrt(jnp.sum(e * e, axis=1, keepdims=True))
    ent = e / jnp.maximum(nrm, 1e-12)
    dot = functools.partial(jnp.dot, preferred_element_type=F32)
    ue = dot(ent, we_ref[...])
    pad = jnp.zeros((N_NODES, TW - 2 * D1), F32)
    tsrc_ref[...] = jnp.concatenate([dot(ent, asrc_ref[...]), ue, pad], axis=1)
    tdst_ref[...] = jnp.concatenate([dot(ent, adst_ref[...]), ue, pad], axis=1)
    ew_ref[...] = dot(ent, went_ref[...])
    rel = rel_ref[...]
    re_ref[...] = dot(rel, arel_ref[...])
    r1p_ref[...] = dot(dot(rel, w1_ref[...]), oar_ref[...])


def _k1(ent_emb, asrc, adst, we, went, rel, arel, w1, oar):
    outs = (
        jax.ShapeDtypeStruct((N_NODES, TW), F32),
        jax.ShapeDtypeStruct((N_NODES, TW), F32),
        jax.ShapeDtypeStruct((N_NODES, D1), F32),
        jax.ShapeDtypeStruct((N_REL, D1), F32),
        jax.ShapeDtypeStruct((N_REL, D1), F32),
    )
    return _pc(_k1_body, outs)(ent_emb, asrc, adst, we, went, rel, arel, w1,
                               oar)


def _att_edge_body(gs_ref, gd_ref, tf_ref, retab_ref, a2m_ref, hmap_ref,
                   pay_ref):
    em = gs_ref[:, :D1] + gd_ref[:, :D1]
    tf = tf_ref[...]
    oh = jnp.where(
        tf == lax.broadcasted_iota(jnp.int32, (EB, N_REL), 1).astype(F32),
        1.0, 0.0)
    dot = functools.partial(jnp.dot, preferred_element_type=F32)
    em = em + dot(oh, retab_ref[...])
    s = dot(em, a2m_ref[...])
    ee = jnp.exp(-_leaky(s))
    mult = dot(ee, hmap_ref[...])
    nh = ee.shape[1]
    pay_ref[...] = jnp.concatenate(
        [em * mult, ee, jnp.zeros((EB, PW - D1 - nh), F32)], axis=1)


def _att_edge(gs, gd, tf, retab, a2m, hmap):
    nh = a2m.shape[1]
    grid = (NBE,)
    return _pc(
        _att_edge_body,
        jax.ShapeDtypeStruct((N_EDGES, PW), F32),
        grid=grid,
        in_specs=[
            pl.BlockSpec((EB, TW), lambda i: (i, 0)),
            pl.BlockSpec((EB, TW), lambda i: (i, 0)),
            pl.BlockSpec((EB, 1), lambda i: (i, 0)),
            pl.BlockSpec((N_REL, D1), lambda i: (0, 0)),
            pl.BlockSpec((D1, nh), lambda i: (0, 0)),
            pl.BlockSpec((nh, D1), lambda i: (0, 0)),
        ],
        out_specs=pl.BlockSpec((EB, PW), lambda i: (i, 0)),
    )(gs, gd, tf, retab, a2m, hmap)


def _k4_body(acc_ref, oas_ref, oad_ref, xs_ref, xd_ref):
    a = acc_ref[0] + acc_ref[1]
    hp0 = a[:, :NHID] / (a[:, D1:D1 + 1] + 1e-12)
    hp1 = a[:, NHID:D1] / (a[:, D1 + 1:D1 + 2] + 1e-12)
    x = _elu(jnp.concatenate([hp0, hp1], axis=1))
    dot = functools.partial(jnp.dot, preferred_element_type=F32)
    pad = jnp.zeros((NPAD, TW - D1), F32)
    xs_ref[...] = jnp.concatenate([dot(x, oas_ref[...]), pad], axis=1)
    xd_ref[...] = jnp.concatenate([dot(x, oad_ref[...]), pad], axis=1)


def _k4(acc1, oas, oad):
    outs = (jax.ShapeDtypeStruct((NPAD, TW), F32),
            jax.ShapeDtypeStruct((NPAD, TW), F32))
    return _pc(_k4_body, outs)(acc1, oas, oad)


def _k8_body(acc_ref, accm_ref, ew_ref, out_ref):
    a = (acc_ref[0] + acc_ref[1])[:N_NODES]
    hp = a[:, :D1] / (a[:, D1:D1 + 1] + 1e-12)
    m = (accm_ref[0] + accm_ref[1])[:N_NODES]
    mask = jnp.minimum(m[:, 0:1], 1.0)
    oe = ew_ref[...] + mask * _elu(hp)
    nrm = jnp.sqrt(jnp.sum(oe * oe, axis=1, keepdims=True))
    out_ref[...] = oe / jnp.maximum(nrm, 1e-12)


def _k8(acc2, accm, ew):
    return _pc(_k8_body, jax.ShapeDtypeStruct((N_NODES, D1), F32))(
        acc2, accm, ew)


def _conv_t(gs, gd, i32eye, mp, bp):
    dot = functools.partial(jax.lax.dot_general,
                            preferred_element_type=F32)
    h = gs[:, D1:2 * D1]
    t = gd[:, D1:2 * D1]
    hT = dot(i32eye, h, dimension_numbers=(((1,), (1,)), ((), ())))
    kT = dot(mp, t, dimension_numbers=(((1,), (1,)), ((), ()))) + bp
    rows = []
    for o in range(OUT_CH):
        accum = None
        for w in range(FILT_W):
            term = hT[w:w + CONV_W, :] * kT[w * OUT_CH + o:w * OUT_CH + o + 1, :]
            accum = term if accum is None else accum + term
        rows.append(accum)
    crT = jnp.concatenate(rows, axis=0)
    return hT, kT, crT


def _h1_body(gs_ref, gd_ref, i32_ref, mp_ref, bp_ref, s_ref,
             st768_ref, st32_ref, a768_ref, a32_ref):
    i = pl.program_id(0)

    @pl.when(i == 0)
    def _init():
        a768_ref[...] = jnp.zeros_like(a768_ref)
        a32_ref[...] = jnp.zeros_like(a32_ref)

    hT, kT, crT = _conv_t(gs_ref[...], gd_ref[...], i32_ref[...], mp_ref[...],
                          bp_ref[...])
    dot = functools.partial(jax.lax.dot_general, preferred_element_type=F32)
    smat = s_ref[...]
    crj = dot(smat, crT, dimension_numbers=(((1,), (0,)), ((), ())))
    ksumT = jnp.zeros((OUT_CH, EB), F32)
    for w in range(FILT_W):
        ksumT = ksumT + kT[w * OUT_CH:(w + 1) * OUT_CH, :]
    a1 = jnp.sum(crj, axis=1, keepdims=True)
    a2v = jnp.sum(crT * crT, axis=1, keepdims=True)
    x1 = jnp.sum(crj * ksumT, axis=1, keepdims=True)
    k1v = jnp.sum(ksumT, axis=1, keepdims=True)
    k2v = jnp.sum(ksumT * ksumT, axis=1, keepdims=True)
    hs1 = jnp.sum(hT)
    hs2 = jnp.sum(hT * hT)
    riota = lax.broadcasted_iota(jnp.int32, (OUT_CH, 1), 0)
    hcol = jnp.where(riota == 0, hs1, jnp.where(riota == 1, hs2, 0.0))
    a768_ref[...] += jnp.concatenate(
        [jnp.zeros((NCR, 1), F32), a2v], axis=1)
    a32_ref[...] += jnp.concatenate([a1, x1, k1v, k2v, hcol], axis=1)

    @pl.when(i == NBE - 1)
    def _fin():
        st768_ref[...] = a768_ref[...]
        st32_ref[...] = a32_ref[...]


def _h1(gs, gd, i32eye, mp, bp, smat):
    outs = (jax.ShapeDtypeStruct((NCR, 2), F32),
            jax.ShapeDtypeStruct((OUT_CH, 5), F32))
    return _pc(
        _h1_body, outs,
        grid=(NBE,),
        in_specs=[
            pl.BlockSpec((EB, TW), lambda i: (i, 0)),
            pl.BlockSpec((EB, TW), lambda i: (i, 0)),
            pl.BlockSpec((D1, D1), lambda i: (0, 0)),
            pl.BlockSpec((FILT_W * OUT_CH, D1), lambda i: (0, 0)),
            pl.BlockSpec((FILT_W * OUT_CH, 1), lambda i: (0, 0)),
            pl.BlockSpec((OUT_CH, NCR), lambda i: (0, 0)),
        ],
        out_specs=(pl.BlockSpec((NCR, 2), lambda i: (0, 0)),
                   pl.BlockSpec((OUT_CH, 5), lambda i: (0, 0))),
        scratch_shapes=[pltpu.VMEM((NCR, 2), F32),
                        pltpu.VMEM((OUT_CH, 5), F32)],
    )(gs, gd, i32eye, mp, bp, smat)


def _kh0_body(st768_ref, st32_ref, s_ref, fcwt_ref, fcbt_ref, bnp_ref, r_ref,
              eye_ref, w1e_ref, w2e_ref, cv_ref):
    dot = functools.partial(jax.lax.dot_general, preferred_element_type=F32)
    smat = s_ref[...]
    a2 = dot(smat, st768_ref[:, 1:2], dimension_numbers=(((1,), (0,)), ((), ())))
    st32 = st32_ref[...]
    a1 = st32[:, 0:1]
    x1 = st32[:, 1:2]
    k1v = st32[:, 2:3]
    k2v = st32[:, 3:4]
    riota = lax.broadcasted_iota(jnp.int32, (OUT_CH, 1), 0)
    hs1 = jnp.sum(jnp.where(riota == 0, st32[:, 4:5], 0.0))
    hs2 = jnp.sum(jnp.where(riota == 1, st32[:, 4:5], 0.0))
    nh = float(N_EDGES * D1)
    mu0 = hs1 / nh
    v0 = hs2 / nh - mu0 * mu0
    bnp = bnp_ref[...]
    g1 = bnp[:, 0:1]
    b1 = bnp[:, 1:2]
    g0 = jnp.sum(jnp.where(riota == 0, bnp[:, 2:3], 0.0))
    b0 = jnp.sum(jnp.where(riota == 0, bnp[:, 3:4], 0.0))
    alpha = g0 / jnp.sqrt(v0 + EPS)
    beta0 = b0 - alpha * mu0
    m1 = float(N_EDGES * CONV_W)
    s1 = alpha * a1 + CONV_W * beta0 * k1v
    s2 = (alpha * alpha * a2 + 2.0 * alpha * beta0 * x1
          + CONV_W * beta0 * beta0 * k2v)
    mu1 = s1 / m1
    v1 = s2 / m1 - mu1 * mu1
    sc = g1 / jnp.sqrt(v1 + EPS)
    c1 = b1 - sc * mu1
    srep = alpha * dot(sc, r_ref[...],
                       dimension_numbers=(((0,), (0,)), ((), ())))
    fcwt = fcwt_ref[...]
    w1e_ref[...] = fcwt * srep
    wjt = dot(fcwt, r_ref[...],
              dimension_numbers=(((1,), (1,)), ((), ())))
    bsc = dot(beta0 * sc, eye_ref[...],
              dimension_numbers=(((0,), (0,)), ((), ())))
    w2e_ref[...] = wjt * bsc
    cv_ref[...] = dot(wjt, c1,
                      dimension_numbers=(((1,), (0,)), ((), ()))) + fcbt_ref[...]


def _kh0(st768, st32, smat, fcwt, fcbt, bnp, rmat, eye32):
    outs = (jax.ShapeDtypeStruct((D1, NCR), F32),
            jax.ShapeDtypeStruct((D1, D1), F32),
            jax.ShapeDtypeStruct((D1, 1), F32))
    return _pc(_kh0_body, outs)(st768, st32, smat, fcwt, fcbt, bnp, rmat,
                                eye32)


def _h2_body(gs_ref, gd_ref, i32_ref, mp_ref, bp_ref, w1e_ref, w2e_ref,
             cv_ref, outT_ref, st2_ref, acc_ref):
    i = pl.program_id(0)

    @pl.when(i == 0)
    def _init():
        acc_ref[...] = jnp.zeros_like(acc_ref)

    _, kT, crT = _conv_t(gs_ref[...], gd_ref[...], i32_ref[...], mp_ref[...],
                         bp_ref[...])
    ksumT = jnp.zeros((OUT_CH, EB), F32)
    for w in range(FILT_W):
        ksumT = ksumT + kT[w * OUT_CH:(w + 1) * OUT_CH, :]
    dot = functools.partial(jax.lax.dot_general, preferred_element_type=F32)
    outT = (dot(w1e_ref[...], crT, dimension_numbers=(((1,), (0,)), ((), ())))
            + dot(w2e_ref[...], ksumT,
                  dimension_numbers=(((1,), (0,)), ((), ())))
            + cv_ref[...])
    outT_ref[...] = outT
    s1 = jnp.sum(outT, axis=1, keepdims=True)
    s2 = jnp.sum(outT * outT, axis=1, keepdims=True)
    acc_ref[...] += jnp.concatenate([s1, s2], axis=1)

    @pl.when(i == NBE - 1)
    def _fin():
        st2_ref[...] = acc_ref[...]


def _h2(gs, gd, i32eye, mp, bp, w1e, w2e, cv):
    outs = (jax.ShapeDtypeStruct((D1, N_EDGES), F32),
            jax.ShapeDtypeStruct((D1, 2), F32))
    return _pc(
        _h2_body, outs,
        grid=(NBE,),
        in_specs=[
            pl.BlockSpec((EB, TW), lambda i: (i, 0)),
            pl.BlockSpec((EB, TW), lambda i: (i, 0)),
            pl.BlockSpec((D1, D1), lambda i: (0, 0)),
            pl.BlockSpec((FILT_W * OUT_CH, D1), lambda i: (0, 0)),
            pl.BlockSpec((FILT_W * OUT_CH, 1), lambda i: (0, 0)),
            pl.BlockSpec((D1, NCR), lambda i: (0, 0)),
            pl.BlockSpec((D1, D1), lambda i: (0, 0)),
            pl.BlockSpec((D1, 1), lambda i: (0, 0)),
        ],
        out_specs=(pl.BlockSpec((D1, EB), lambda i: (0, i)),
                   pl.BlockSpec((D1, 2), lambda i: (0, 0))),
        scratch_shapes=[pltpu.VMEM((D1, 2), F32)],
    )(gs, gd, i32eye, mp, bp, w1e, w2e, cv)


def _h3_body(outT_ref, st2_ref, tf_ref, bn2_ref, i64_ref, rel_ref,
             racc_ref, cacc_ref):
    i = pl.program_id(0)

    @pl.when(i == 0)
    def _init():
        racc_ref[...] = jnp.zeros_like(racc_ref)
        cacc_ref[...] = jnp.zeros_like(cacc_ref)

    st2 = st2_ref[...]
    mu2 = st2[:, 0:1] / float(N_EDGES)
    v2 = st2[:, 1:2] / float(N_EDGES) - mu2 * mu2
    sc2 = bn2_ref[:, 0:1] / jnp.sqrt(v2 + EPS)
    b2 = bn2_ref[:, 1:2]
    val = jnp.maximum((outT_ref[...] - mu2) * sc2 + b2, 0.0)
    tf = tf_ref[...]
    oh = jnp.where(
        tf == lax.broadcasted_iota(jnp.int32, (EB, N_REL), 1).astype(F32),
        1.0, 0.0)
    dot = functools.partial(jax.lax.dot_general, preferred_element_type=F32)
    racc_ref[...] += dot(val, oh, dimension_numbers=(((1,), (0,)), ((), ())))
    cacc_ref[0:1, :] += jnp.sum(oh, axis=0, keepdims=True)

    @pl.when(i == NBE - 1)
    def _fin():
        cnt = jnp.maximum(cacc_ref[0:1, :], 1.0)
        rm = racc_ref[...] / cnt
        nrm = jnp.sqrt(jnp.sum(rm * rm, axis=0, keepdims=True))
        rn = rm / jnp.maximum(nrm, 1e-12)
        rel_ref[...] = dot(i64_ref[...], rn,
                           dimension_numbers=(((1,), (1,)), ((), ())))


def _h3(outT, st2, tf, bn2, i64eye):
    return _pc(
        _h3_body,
        jax.ShapeDtypeStruct((N_REL, D1), F32),
        grid=(NBE,),
        in_specs=[
            pl.BlockSpec((D1, EB), lambda i: (0, i)),
            pl.BlockSpec((D1, 2), lambda i: (0, 0)),
            pl.BlockSpec((EB, 1), lambda i: (i, 0)),
            pl.BlockSpec((D1, 2), lambda i: (0, 0)),
            pl.BlockSpec((N_REL, N_REL), lambda i: (0, 0)),
        ],
        out_specs=pl.BlockSpec((N_REL, D1), lambda i: (0, 0)),
        scratch_shapes=[pltpu.VMEM((D1, N_REL), F32),
                        pltpu.VMEM((8, N_REL), F32)],
    )(outT, st2, tf, bn2, i64eye)


def _sc_gather(table, idx):
    n, d = table.shape
    e = idx.shape[0]
    nch = e // CH
    base_n = nch // 32
    rem = nch % 32
    mesh = plsc.VectorSubcoreMesh(core_axis_name="c", subcore_axis_name="s")

    @functools.partial(
        pl.kernel, mesh=mesh,
        out_type=jax.ShapeDtypeStruct((e, d), F32),
        scratch_types=[pltpu.VMEM((CH,), jnp.int32),
                       pltpu.VMEM((CH, d), F32),
                       pltpu.SemaphoreType.DMA],
    )
    def k(table_hbm, idx_hbm, out_hbm, idx_v, rows_v, sem):
        wid = lax.axis_index("s") * 2 + lax.axis_index("c")

        def chunk(c):
            off = c * CH
            pltpu.sync_copy(idx_hbm.at[pl.ds(off, CH)], idx_v)
            pltpu.async_copy(table_hbm.at[idx_v], rows_v, sem).wait()
            pltpu.sync_copy(rows_v, out_hbm.at[pl.ds(off, CH)])

        def body(t, carry):
            chunk(t * 32 + wid)
            return carry

        lax.fori_loop(0, base_n, body, 0)
        if rem:
            @pl.when(wid < rem)
            def _tail():
                chunk(base_n * 32 + wid)

    return k(table, idx)


def _sc_scatter_add(vals, idx, n):
    e, d = vals.shape
    nch = e // CH
    base_n = nch // 32
    rem = nch % 32
    rows_t = n // 16
    mesh = plsc.VectorSubcoreMesh(core_axis_name="c", subcore_axis_name="s")
    zeros = jnp.zeros((n, d), F32)

    @functools.partial(
        pl.kernel, mesh=mesh,
        out_type=jax.ShapeDtypeStruct((2, n, d), F32),
        scratch_types=[pltpu.VMEM((CH,), jnp.int32),
                       pltpu.VMEM((CH, d), F32),
                       pltpu.VMEM_SHARED((n, d), F32)],
    )
    def k(vals_hbm, idx_hbm, zero_hbm, out_hbm, idx_v, rows_v, acc_sh):
        cid = lax.axis_index("c")
        sid = lax.axis_index("s")
        wid = sid * 2 + cid
        pltpu.sync_copy(zero_hbm.at[pl.ds(sid * rows_t, rows_t)],
                        acc_sh.at[pl.ds(sid * rows_t, rows_t)])
        plsc.subcore_barrier()

        def chunk(c):
            off = c * CH
            pltpu.sync_copy(idx_hbm.at[pl.ds(off, CH)], idx_v)
            pltpu.sync_copy(vals_hbm.at[pl.ds(off, CH)], rows_v)
            pltpu.sync_copy(rows_v, acc_sh.at[idx_v], add=True)

        def body(t, carry):
            chunk(t * 32 + wid)
            return carry

        lax.fori_loop(0, base_n, body, 0)
        if rem:
            @pl.when(wid < rem)
            def _tail():
                chunk(base_n * 32 + wid)

        plsc.subcore_barrier()
        pltpu.sync_copy(acc_sh.at[pl.ds(sid * rows_t, rows_t)],
                        out_hbm.at[cid].at[pl.ds(sid * rows_t, rows_t)])

    return k(vals, idx, zeros)


def _emu_scatter(vals, idx, n):
    out = jnp.zeros((2, n, vals.shape[1]), F32)
    return out.at[0].set(jax.ops.segment_sum(vals, idx, num_segments=n))


def kernel(edge_list, edge_type, batch_inputs, params):
    p = params
    src = edge_list[0].astype(jnp.int32)
    dst = edge_list[1].astype(jnp.int32)
    tf = edge_type.astype(F32).reshape(N_EDGES, 1)

    aa = p['att_a']
    asrc = jnp.concatenate([aa[0, :, :IN_DIM], aa[1, :, :IN_DIM]], 0).T
    adst = jnp.concatenate([aa[0, :, IN_DIM:2 * IN_DIM],
                            aa[1, :, IN_DIM:2 * IN_DIM]], 0).T
    arel = jnp.concatenate([aa[0, :, 2 * IN_DIM:], aa[1, :, 2 * IN_DIM:]], 0).T
    a2 = p['att_a2']
    a2m = jnp.zeros((D1, NHEADS), F32)
    a2m = a2m.at[:NHID, 0].set(a2[0, 0]).at[NHID:, 1].set(a2[1, 0])
    hmap = jnp.zeros((NHEADS, D1), F32)
    hmap = hmap.at[0, :NHID].set(1.0).at[1, NHID:].set(1.0)
    oa = p['out_a']
    oas, oad, oar = oa[:, :D1].T, oa[:, D1:2 * D1].T, oa[:, 2 * D1:].T
    oa2m = p['out_a2'].T
    h1map = jnp.ones((1, D1), F32)
    mp = p['fc1_w'].T.reshape(OUT_CH, FILT_W, D1).transpose(1, 0, 2) \
        .reshape(FILT_W * OUT_CH, D1)
    bp = p['fc1_b'].reshape(OUT_CH, FILT_W).T.reshape(FILT_W * OUT_CH, 1)
    i32eye = jnp.eye(D1, dtype=F32)
    i64eye = jnp.eye(N_REL, dtype=F32)
    smat = jnp.kron(jnp.eye(OUT_CH, dtype=F32), jnp.ones((1, CONV_W), F32))
    rmat = smat
    fcwt = p['fc_w'].T
    fcbt = p['fc_b'].reshape(D1, 1)
    bnp = jnp.concatenate([
        p['bn1_g'].reshape(OUT_CH, 1), p['bn1_b'].reshape(OUT_CH, 1),
        jnp.full((OUT_CH, 1), p['bn0_g'][0]),
        jnp.full((OUT_CH, 1), p['bn0_b'][0])], axis=1)
    bn2 = jnp.concatenate([p['bn2_g'].reshape(D1, 1),
                           p['bn2_b'].reshape(D1, 1)], axis=1)

    tsrc, tdst, ew, re, r1p = _k1(
        p['entity_embeddings'], asrc, adst, p['W_E'], p['W_entities'],
        p['relation_embeddings'], arel, p['W_1'], oar)

    gs = _sc_gather(tsrc, src)
    gd = _sc_gather(tdst, dst)

    pay1 = _att_edge(gs, gd, tf, re, a2m, hmap)
    acc1 = _sc_scatter_add(pay1, src, NPAD)
    xs, xd = _k4(acc1, oas, oad)

    gs2 = _sc_gather(xs, src)
    gd2 = _sc_gather(xd, dst)
    pay2 = _att_edge(gs2, gd2, tf, r1p, oa2m, h1map)
    acc2 = _sc_scatter_add(pay2, src, NPAD)

    ones = jnp.ones((BATCH, TW), F32)
    tgt = batch_inputs[:, 2].astype(jnp.int32)
    accm = _sc_scatter_add(ones, tgt, NPAD)

    out_entity = _k8(acc2, accm, ew)

    st768, st32 = _h1(gs, gd, i32eye, mp, bp, smat)
    w1e, w2e, cv = _kh0(st768, st32, smat, fcwt, fcbt, bnp, rmat, i32eye)
    outT, st2 = _h2(gs, gd, i32eye, mp, bp, w1e, w2e, cv)
    out_relation = _h3(outT, st2, tf, bn2, i64eye)

    return (out_entity, out_relation)


# speedup vs baseline: 3.8322x; 1.2710x over previous
import functools

import jax
import jax.numpy as jnp
import numpy as np
from jax import lax
from jax.experimental import pallas as pl
from jax.experimental.pallas import tpu as pltpu
from jax.experimental.pallas import tpu_sc as plsc

N_NODES = 10000
N_REL = 64
IN_DIM = 128
REL_DIM = 64
NHID = 16
NHEADS = 2
D1 = NHID * NHEADS
N_EDGES = 160000
BATCH = 8192
ALPHA = 0.2
EPS = 1e-5
FILT_W = 9
OUT_CH = 32
CONV_W = D1 - FILT_W + 1
NCR = OUT_CH * CONV_W

EB = 1280
NBE = N_EDGES // EB
PW = 128
TW = 128
NPAD = 10240
CH = 128
F32 = jnp.float32

_INTERPRET = False


def _leaky(x):
    return jnp.where(x > 0, x, ALPHA * x)


def _elu(x):
    return jnp.where(x > 0, x, jnp.exp(jnp.minimum(x, 0.0)) - 1.0)


def _pc(body, out_shape, **kw):
    return pl.pallas_call(body, out_shape=out_shape, interpret=_INTERPRET, **kw)


def _k1_body(e_ref, asrc_ref, adst_ref, we_ref, went_ref, rel_ref, arel_ref,
             w1_ref, oar_ref, tsrc_ref, tdst_ref, ew_ref, re_ref,
             r1p_ref):
    e = e_ref[...]
    nrm = jnp.sqrt(jnp.sum(e * e, axis=1, keepdims=True))
    ent = e / jnp.maximum(nrm, 1e-12)
    dot = functools.partial(jnp.dot, preferred_element_type=F32)
    ue = dot(ent, we_ref[...])
    pad = jnp.zeros((N_NODES, TW - 2 * D1), F32)
    tsrc_ref[...] = jnp.concatenate([dot(ent, asrc_ref[...]), ue, pad], axis=1)
    tdst_ref[...] = jnp.concatenate([dot(ent, adst_ref[...]), ue, pad], axis=1)
    ew_ref[...] = dot(ent, went_ref[...])
    rel = rel_ref[...]
    re_ref[...] = dot(rel, arel_ref[...])
    r1p_ref[...] = dot(dot(rel, w1_ref[...]), oar_ref[...])


def _k1(ent_emb, asrc, adst, we, went, rel, arel, w1, oar):
    outs = (
        jax.ShapeDtypeStruct((N_NODES, TW), F32),
        jax.ShapeDtypeStruct((N_NODES, TW), F32),
        jax.ShapeDtypeStruct((N_NODES, D1), F32),
        jax.ShapeDtypeStruct((N_REL, D1), F32),
        jax.ShapeDtypeStruct((N_REL, D1), F32),
    )
    return _pc(_k1_body, outs)(ent_emb, asrc, adst, we, went, rel, arel, w1,
                               oar)


def _att_edge_body(gs_ref, gd_ref, tf_ref, retab_ref, a2m_ref, hmap_ref,
                   pay_ref):
    em = gs_ref[:, :D1] + gd_ref[:, :D1]
    tf = tf_ref[...]
    oh = jnp.where(
        tf == lax.broadcasted_iota(jnp.int32, (EB, N_REL), 1).astype(F32),
        1.0, 0.0)
    dot = functools.partial(jnp.dot, preferred_element_type=F32)
    em = em + dot(oh, retab_ref[...])
    s = dot(em, a2m_ref[...])
    ee = jnp.exp(-_leaky(s))
    mult = dot(ee, hmap_ref[...])
    nh = ee.shape[1]
    pay_ref[...] = jnp.concatenate(
        [em * mult, ee, jnp.zeros((EB, PW - D1 - nh), F32)], axis=1)


def _att_edge(gs, gd, tf, retab, a2m, hmap):
    nh = a2m.shape[1]
    grid = (NBE,)
    return _pc(
        _att_edge_body,
        jax.ShapeDtypeStruct((N_EDGES, PW), F32),
        grid=grid,
        in_specs=[
            pl.BlockSpec((EB, TW), lambda i: (i, 0)),
            pl.BlockSpec((EB, TW), lambda i: (i, 0)),
            pl.BlockSpec((EB, 1), lambda i: (i, 0)),
            pl.BlockSpec((N_REL, D1), lambda i: (0, 0)),
            pl.BlockSpec((D1, nh), lambda i: (0, 0)),
            pl.BlockSpec((nh, D1), lambda i: (0, 0)),
        ],
        out_specs=pl.BlockSpec((EB, PW), lambda i: (i, 0)),
    )(gs, gd, tf, retab, a2m, hmap)


def _k4_body(acc_ref, oas_ref, oad_ref, xs_ref, xd_ref):
    a = acc_ref[0] + acc_ref[1]
    hp0 = a[:, :NHID] / (a[:, D1:D1 + 1] + 1e-12)
    hp1 = a[:, NHID:D1] / (a[:, D1 + 1:D1 + 2] + 1e-12)
    x = _elu(jnp.concatenate([hp0, hp1], axis=1))
    dot = functools.partial(jnp.dot, preferred_element_type=F32)
    pad = jnp.zeros((NPAD, TW - D1), F32)
    xs_ref[...] = jnp.concatenate([dot(x, oas_ref[...]), pad], axis=1)
    xd_ref[...] = jnp.concatenate([dot(x, oad_ref[...]), pad], axis=1)


def _k4(acc1, oas, oad):
    outs = (jax.ShapeDtypeStruct((NPAD, TW), F32),
            jax.ShapeDtypeStruct((NPAD, TW), F32))
    return _pc(_k4_body, outs)(acc1, oas, oad)


def _k8_body(acc_ref, accm_ref, ew_ref, out_ref):
    a = (acc_ref[0] + acc_ref[1])[:N_NODES]
    hp = a[:, :D1] / (a[:, D1:D1 + 1] + 1e-12)
    m = (accm_ref[0] + accm_ref[1])[:N_NODES]
    mask = jnp.minimum(m[:, 0:1], 1.0)
    oe = ew_ref[...] + mask * _elu(hp)
    nrm = jnp.sqrt(jnp.sum(oe * oe, axis=1, keepdims=True))
    out_ref[...] = oe / jnp.maximum(nrm, 1e-12)


def _k8(acc2, accm, ew):
    return _pc(_k8_body, jax.ShapeDtypeStruct((N_NODES, D1), F32))(
        acc2, accm, ew)


def _conv_t(gs, gd, i32eye, mp, bp):
    dot = functools.partial(jax.lax.dot_general,
                            preferred_element_type=F32)
    h = gs[:, D1:2 * D1]
    t = gd[:, D1:2 * D1]
    hT = dot(i32eye, h, dimension_numbers=(((1,), (1,)), ((), ())))
    kT = dot(mp, t, dimension_numbers=(((1,), (1,)), ((), ()))) + bp
    rows = []
    for o in range(OUT_CH):
        accum = None
        for w in range(FILT_W):
            term = hT[w:w + CONV_W, :] * kT[w * OUT_CH + o:w * OUT_CH + o + 1, :]
            accum = term if accum is None else accum + term
        rows.append(accum)
    crT = jnp.concatenate(rows, axis=0)
    return hT, kT, crT


def _h1_body(gs_ref, gd_ref, i32_ref, mp_ref, bp_ref, s_ref,
             st768_ref, st32_ref, a768_ref, a32_ref):
    i = pl.program_id(0)

    @pl.when(i == 0)
    def _init():
        a768_ref[...] = jnp.zeros_like(a768_ref)
        a32_ref[...] = jnp.zeros_like(a32_ref)

    hT, kT, crT = _conv_t(gs_ref[...], gd_ref[...], i32_ref[...], mp_ref[...],
                          bp_ref[...])
    dot = functools.partial(jax.lax.dot_general, preferred_element_type=F32)
    smat = s_ref[...]
    crj = dot(smat, crT, dimension_numbers=(((1,), (0,)), ((), ())))
    ksumT = jnp.zeros((OUT_CH, EB), F32)
    for w in range(FILT_W):
        ksumT = ksumT + kT[w * OUT_CH:(w + 1) * OUT_CH, :]
    a1 = jnp.sum(crj, axis=1, keepdims=True)
    a2v = jnp.sum(crT * crT, axis=1, keepdims=True)
    x1 = jnp.sum(crj * ksumT, axis=1, keepdims=True)
    k1v = jnp.sum(ksumT, axis=1, keepdims=True)
    k2v = jnp.sum(ksumT * ksumT, axis=1, keepdims=True)
    hs1 = jnp.sum(hT)
    hs2 = jnp.sum(hT * hT)
    riota = lax.broadcasted_iota(jnp.int32, (OUT_CH, 1), 0)
    hcol = jnp.where(riota == 0, hs1, jnp.where(riota == 1, hs2, 0.0))
    a768_ref[...] += jnp.concatenate(
        [jnp.zeros((NCR, 1), F32), a2v], axis=1)
    a32_ref[...] += jnp.concatenate([a1, x1, k1v, k2v, hcol], axis=1)

    @pl.when(i == NBE - 1)
    def _fin():
        st768_ref[...] = a768_ref[...]
        st32_ref[...] = a32_ref[...]


def _h1(gs, gd, i32eye, mp, bp, smat):
    outs = (jax.ShapeDtypeStruct((NCR, 2), F32),
            jax.ShapeDtypeStruct((OUT_CH, 5), F32))
    return _pc(
        _h1_body, outs,
        grid=(NBE,),
        in_specs=[
            pl.BlockSpec((EB, TW), lambda i: (i, 0)),
            pl.BlockSpec((EB, TW), lambda i: (i, 0)),
            pl.BlockSpec((D1, D1), lambda i: (0, 0)),
            pl.BlockSpec((FILT_W * OUT_CH, D1), lambda i: (0, 0)),
            pl.BlockSpec((FILT_W * OUT_CH, 1), lambda i: (0, 0)),
            pl.BlockSpec((OUT_CH, NCR), lambda i: (0, 0)),
        ],
        out_specs=(pl.BlockSpec((NCR, 2), lambda i: (0, 0)),
                   pl.BlockSpec((OUT_CH, 5), lambda i: (0, 0))),
        scratch_shapes=[pltpu.VMEM((NCR, 2), F32),
                        pltpu.VMEM((OUT_CH, 5), F32)],
    )(gs, gd, i32eye, mp, bp, smat)


def _kh0_body(st768_ref, st32_ref, s_ref, fr_ref, f1s_ref, jr_ref, fcbr_ref,
              bnp_ref, gm_ref):
    dot = functools.partial(jax.lax.dot_general, preferred_element_type=F32)
    smat = s_ref[...]
    a2 = dot(smat, st768_ref[:, 1:2], dimension_numbers=(((1,), (0,)), ((), ())))
    st32 = st32_ref[...]
    a1 = st32[:, 0:1]
    x1 = st32[:, 1:2]
    k1v = st32[:, 2:3]
    k2v = st32[:, 3:4]
    riota = lax.broadcasted_iota(jnp.int32, (OUT_CH, 1), 0)
    hs1 = jnp.sum(jnp.where(riota == 0, st32[:, 4:5], 0.0))
    hs2 = jnp.sum(jnp.where(riota == 1, st32[:, 4:5], 0.0))
    nh = float(N_EDGES * D1)
    mu0 = hs1 / nh
    v0 = hs2 / nh - mu0 * mu0
    bnp = bnp_ref[...]
    g1 = bnp[:, 0:1]
    b1 = bnp[:, 1:2]
    g0 = jnp.sum(jnp.where(riota == 0, bnp[:, 2:3], 0.0))
    b0 = jnp.sum(jnp.where(riota == 0, bnp[:, 3:4], 0.0))
    alpha = g0 / jnp.sqrt(v0 + EPS)
    beta0 = b0 - alpha * mu0
    m1 = float(N_EDGES * CONV_W)
    s1 = alpha * a1 + CONV_W * beta0 * k1v
    s2 = (alpha * alpha * a2 + 2.0 * alpha * beta0 * x1
          + CONV_W * beta0 * beta0 * k2v)
    mu1 = s1 / m1
    v1 = s2 / m1 - mu1 * mu1
    sc = g1 / jnp.sqrt(v1 + EPS)
    c1 = b1 - sc * mu1
    fr_sc = fr_ref[...] * sc
    gshift = jnp.zeros((D1 + 1, D1 * D1), F32)
    colsum = jnp.zeros((D1 + 1, NCR), F32)
    for w in range(FILT_W):
        term = dot(f1s_ref[pl.ds(w * (D1 + 1), D1 + 1), :], fr_sc,
                   dimension_numbers=(((1,), (0,)), ((), ())))
        colsum = colsum + term
        gshift = gshift + jnp.pad(
            term, ((0, 0), (D1 * w, D1 * D1 - NCR - D1 * w)))
    jred = dot(colsum, jr_ref[...],
               dimension_numbers=(((1,), (0,)), ((), ())))
    wjt2 = dot(fr_ref[...], jr_ref[...],
               dimension_numbers=(((1,), (0,)), ((), ())))
    cvrow = dot(c1, wjt2,
                dimension_numbers=(((0,), (0,)), ((), ()))) + fcbr_ref[...]
    citer = lax.broadcasted_iota(jnp.int32, (D1 + 1, 1), 0)
    cvblk = jnp.where(citer == D1, 1.0, 0.0) * cvrow
    gm_ref[...] = jnp.concatenate(
        [alpha * gshift, beta0 * jred + cvblk], axis=1)


def _kh0(st768, st32, smat, fr, f1s, jr, fcbr, bnp):
    return _pc(_kh0_body,
               jax.ShapeDtypeStruct((D1 + 1, D1 * (D1 + 1)), F32))(
        st768, st32, smat, fr, f1s, jr, fcbr, bnp)


def _h2_body(gs_ref, gd_ref, i32_ref, gm2_ref, outT_ref, st2_ref, acc_ref):
    i = pl.program_id(0)

    @pl.when(i == 0)
    def _init():
        acc_ref[...] = jnp.zeros_like(acc_ref)

    dot = functools.partial(jax.lax.dot_general, preferred_element_type=F32)
    h = gs_ref[:, D1:2 * D1]
    t = gd_ref[:, D1:2 * D1]
    eye = i32_ref[...]
    hT = dot(eye, h, dimension_numbers=(((1,), (1,)), ((), ())))
    tT = dot(eye, t, dimension_numbers=(((1,), (1,)), ((), ())))
    ones = jnp.ones((1, EB), F32)
    haT = jnp.concatenate([hT, ones], axis=0)
    taT = jnp.concatenate([tT, ones], axis=0)
    pt = jnp.concatenate(
        [taT * haT[m:m + 1, :] for m in range(D1 + 1)], axis=0)
    outT = dot(gm2_ref[...], pt,
               dimension_numbers=(((0,), (0,)), ((), ())))
    outT_ref[...] = outT
    s1 = jnp.sum(outT, axis=1, keepdims=True)
    s2 = jnp.sum(outT * outT, axis=1, keepdims=True)
    acc_ref[...] += jnp.concatenate([s1, s2], axis=1)

    @pl.when(i == NBE - 1)
    def _fin():
        st2_ref[...] = acc_ref[...]


def _h2(gs, gd, i32eye, gm2):
    outs = (jax.ShapeDtypeStruct((D1, N_EDGES), F32),
            jax.ShapeDtypeStruct((D1, 2), F32))
    return _pc(
        _h2_body, outs,
        grid=(NBE,),
        in_specs=[
            pl.BlockSpec((EB, TW), lambda i: (i, 0)),
            pl.BlockSpec((EB, TW), lambda i: (i, 0)),
            pl.BlockSpec((D1, D1), lambda i: (0, 0)),
            pl.BlockSpec(((D1 + 1) * (D1 + 1), D1), lambda i: (0, 0)),
        ],
        out_specs=(pl.BlockSpec((D1, EB), lambda i: (0, i)),
                   pl.BlockSpec((D1, 2), lambda i: (0, 0))),
        scratch_shapes=[pltpu.VMEM((D1, 2), F32)],
    )(gs, gd, i32eye, gm2)


def _h3_body(outT_ref, st2_ref, tf_ref, bn2_ref, i64_ref, rel_ref,
             racc_ref, cacc_ref):
    i = pl.program_id(0)

    @pl.when(i == 0)
    def _init():
        racc_ref[...] = jnp.zeros_like(racc_ref)
        cacc_ref[...] = jnp.zeros_like(cacc_ref)

    st2 = st2_ref[...]
    mu2 = st2[:, 0:1] / float(N_EDGES)
    v2 = st2[:, 1:2] / float(N_EDGES) - mu2 * mu2
    sc2 = bn2_ref[:, 0:1] / jnp.sqrt(v2 + EPS)
    b2 = bn2_ref[:, 1:2]
    val = jnp.maximum((outT_ref[...] - mu2) * sc2 + b2, 0.0)
    tf = tf_ref[...]
    oh = jnp.where(
        tf == lax.broadcasted_iota(jnp.int32, (EB, N_REL), 1).astype(F32),
        1.0, 0.0)
    dot = functools.partial(jax.lax.dot_general, preferred_element_type=F32)
    racc_ref[...] += dot(val, oh, dimension_numbers=(((1,), (0,)), ((), ())))
    cacc_ref[0:1, :] += jnp.sum(oh, axis=0, keepdims=True)

    @pl.when(i == NBE - 1)
    def _fin():
        cnt = jnp.maximum(cacc_ref[0:1, :], 1.0)
        rm = racc_ref[...] / cnt
        nrm = jnp.sqrt(jnp.sum(rm * rm, axis=0, keepdims=True))
        rn = rm / jnp.maximum(nrm, 1e-12)
        rel_ref[...] = dot(i64_ref[...], rn,
                           dimension_numbers=(((1,), (1,)), ((), ())))


def _h3(outT, st2, tf, bn2, i64eye):
    return _pc(
        _h3_body,
        jax.ShapeDtypeStruct((N_REL, D1), F32),
        grid=(NBE,),
        in_specs=[
            pl.BlockSpec((D1, EB), lambda i: (0, i)),
            pl.BlockSpec((D1, 2), lambda i: (0, 0)),
            pl.BlockSpec((EB, 1), lambda i: (i, 0)),
            pl.BlockSpec((D1, 2), lambda i: (0, 0)),
            pl.BlockSpec((N_REL, N_REL), lambda i: (0, 0)),
        ],
        out_specs=pl.BlockSpec((N_REL, D1), lambda i: (0, 0)),
        scratch_shapes=[pltpu.VMEM((D1, N_REL), F32),
                        pltpu.VMEM((8, N_REL), F32)],
    )(outT, st2, tf, bn2, i64eye)


def _sc_gather(table, idx):
    n, d = table.shape
    e = idx.shape[0]
    nch = e // CH
    base_n = nch // 32
    rem = nch % 32
    mesh = plsc.VectorSubcoreMesh(core_axis_name="c", subcore_axis_name="s")

    @functools.partial(
        pl.kernel, mesh=mesh,
        out_type=jax.ShapeDtypeStruct((e, d), F32),
        scratch_types=[pltpu.VMEM((CH,), jnp.int32),
                       pltpu.VMEM((CH, d), F32),
                       pltpu.SemaphoreType.DMA],
    )
    def k(table_hbm, idx_hbm, out_hbm, idx_v, rows_v, sem):
        wid = lax.axis_index("s") * 2 + lax.axis_index("c")

        def chunk(c):
            off = c * CH
            pltpu.sync_copy(idx_hbm.at[pl.ds(off, CH)], idx_v)
            pltpu.async_copy(table_hbm.at[idx_v], rows_v, sem).wait()
            pltpu.sync_copy(rows_v, out_hbm.at[pl.ds(off, CH)])

        def body(t, carry):
            chunk(t * 32 + wid)
            return carry

        lax.fori_loop(0, base_n, body, 0)
        if rem:
            @pl.when(wid < rem)
            def _tail():
                chunk(base_n * 32 + wid)

    return k(table, idx)


def _sc_scatter_add(vals, idx, n):
    e, d = vals.shape
    nch = e // CH
    base_n = nch // 32
    rem = nch % 32
    rows_t = n // 16
    mesh = plsc.VectorSubcoreMesh(core_axis_name="c", subcore_axis_name="s")
    zeros = jnp.zeros((n, d), F32)

    @functools.partial(
        pl.kernel, mesh=mesh,
        out_type=jax.ShapeDtypeStruct((2, n, d), F32),
        scratch_types=[pltpu.VMEM((CH,), jnp.int32),
                       pltpu.VMEM((CH, d), F32),
                       pltpu.VMEM_SHARED((n, d), F32)],
    )
    def k(vals_hbm, idx_hbm, zero_hbm, out_hbm, idx_v, rows_v, acc_sh):
        cid = lax.axis_index("c")
        sid = lax.axis_index("s")
        wid = sid * 2 + cid
        pltpu.sync_copy(zero_hbm.at[pl.ds(sid * rows_t, rows_t)],
                        acc_sh.at[pl.ds(sid * rows_t, rows_t)])
        plsc.subcore_barrier()

        def chunk(c):
            off = c * CH
            pltpu.sync_copy(idx_hbm.at[pl.ds(off, CH)], idx_v)
            pltpu.sync_copy(vals_hbm.at[pl.ds(off, CH)], rows_v)
            pltpu.sync_copy(rows_v, acc_sh.at[idx_v], add=True)

        def body(t, carry):
            chunk(t * 32 + wid)
            return carry

        lax.fori_loop(0, base_n, body, 0)
        if rem:
            @pl.when(wid < rem)
            def _tail():
                chunk(base_n * 32 + wid)

        plsc.subcore_barrier()
        pltpu.sync_copy(acc_sh.at[pl.ds(sid * rows_t, rows_t)],
                        out_hbm.at[cid].at[pl.ds(sid * rows_t, rows_t)])

    return k(vals, idx, zeros)


def _emu_scatter(vals, idx, n):
    out = jnp.zeros((2, n, vals.shape[1]), F32)
    return out.at[0].set(jax.ops.segment_sum(vals, idx, num_segments=n))


def kernel(edge_list, edge_type, batch_inputs, params):
    p = params
    src = edge_list[0].astype(jnp.int32)
    dst = edge_list[1].astype(jnp.int32)
    tf = edge_type.astype(F32).reshape(N_EDGES, 1)

    aa = p['att_a']
    asrc = jnp.concatenate([aa[0, :, :IN_DIM], aa[1, :, :IN_DIM]], 0).T
    adst = jnp.concatenate([aa[0, :, IN_DIM:2 * IN_DIM],
                            aa[1, :, IN_DIM:2 * IN_DIM]], 0).T
    arel = jnp.concatenate([aa[0, :, 2 * IN_DIM:], aa[1, :, 2 * IN_DIM:]], 0).T
    a2 = p['att_a2']
    a2m = jnp.zeros((D1, NHEADS), F32)
    a2m = a2m.at[:NHID, 0].set(a2[0, 0]).at[NHID:, 1].set(a2[1, 0])
    hmap = jnp.zeros((NHEADS, D1), F32)
    hmap = hmap.at[0, :NHID].set(1.0).at[1, NHID:].set(1.0)
    oa = p['out_a']
    oas, oad, oar = oa[:, :D1].T, oa[:, D1:2 * D1].T, oa[:, 2 * D1:].T
    oa2m = p['out_a2'].T
    h1map = jnp.ones((1, D1), F32)
    mp = p['fc1_w'].T.reshape(OUT_CH, FILT_W, D1).transpose(1, 0, 2) \
        .reshape(FILT_W * OUT_CH, D1)
    bp = p['fc1_b'].reshape(OUT_CH, FILT_W).T.reshape(FILT_W * OUT_CH, 1)
    i32eye = jnp.eye(D1, dtype=F32)
    i64eye = jnp.eye(N_REL, dtype=F32)
    smat = jnp.kron(jnp.eye(OUT_CH, dtype=F32), jnp.ones((1, CONV_W), F32))
    fr = p['fc_w'].reshape(OUT_CH, CONV_W * D1)
    f1aug = jnp.concatenate([p['fc1_w'], p['fc1_b'].reshape(1, -1)], axis=0)
    f1s = f1aug.reshape(D1 + 1, OUT_CH, FILT_W).transpose(2, 0, 1).reshape(
        FILT_W * (D1 + 1), OUT_CH)
    jr = jnp.tile(jnp.eye(D1, dtype=F32), (CONV_W, 1))
    fcbr = p['fc_b'].reshape(1, D1)
    bnp = jnp.concatenate([
        p['bn1_g'].reshape(OUT_CH, 1), p['bn1_b'].reshape(OUT_CH, 1),
        jnp.full((OUT_CH, 1), p['bn0_g'][0]),
        jnp.full((OUT_CH, 1), p['bn0_b'][0])], axis=1)
    bn2 = jnp.concatenate([p['bn2_g'].reshape(D1, 1),
                           p['bn2_b'].reshape(D1, 1)], axis=1)

    tsrc, tdst, ew, re, r1p = _k1(
        p['entity_embeddings'], asrc, adst, p['W_E'], p['W_entities'],
        p['relation_embeddings'], arel, p['W_1'], oar)

    gs = _sc_gather(tsrc, src)
    gd = _sc_gather(tdst, dst)

    pay1 = _att_edge(gs, gd, tf, re, a2m, hmap)
    acc1 = _sc_scatter_add(pay1, src, NPAD)
    xs, xd = _k4(acc1, oas, oad)

    gs2 = _sc_gather(xs, src)
    gd2 = _sc_gather(xd, dst)
    pay2 = _att_edge(gs2, gd2, tf, r1p, oa2m, h1map)
    acc2 = _sc_scatter_add(pay2, src, NPAD)

    ones = jnp.ones((BATCH, TW), F32)
    tgt = batch_inputs[:, 2].astype(jnp.int32)
    accm = _sc_scatter_add(ones, tgt, NPAD)

    out_entity = _k8(acc2, accm, ew)

    st768, st32 = _h1(gs, gd, i32eye, mp, bp, smat)
    gm = _kh0(st768, st32, smat, fr, f1s, jr, fcbr, bnp)
    gm2 = gm.reshape(D1 + 1, D1 + 1, D1).transpose(1, 0, 2).reshape(
        (D1 + 1) * (D1 + 1), D1)
    outT, st2 = _h2(gs, gd, i32eye, gm2)
    out_relation = _h3(outT, st2, tf, bn2, i64eye)

    return (out_entity, out_relation)


# speedup vs baseline: 3.8627x; 1.0080x over previous
import functools

import jax
import jax.numpy as jnp
import numpy as np
from jax import lax
from jax.experimental import pallas as pl
from jax.experimental.pallas import tpu as pltpu
from jax.experimental.pallas import tpu_sc as plsc

N_NODES = 10000
N_REL = 64
IN_DIM = 128
REL_DIM = 64
NHID = 16
NHEADS = 2
D1 = NHID * NHEADS
N_EDGES = 160000
BATCH = 8192
ALPHA = 0.2
EPS = 1e-5
FILT_W = 9
OUT_CH = 32
CONV_W = D1 - FILT_W + 1
NCR = OUT_CH * CONV_W

EB = 1280
NBE = N_EDGES // EB
PW = 128
TW = 128
NPAD = 10240
CH = 128
F32 = jnp.float32

_INTERPRET = False


def _leaky(x):
    return jnp.where(x > 0, x, ALPHA * x)


def _elu(x):
    return jnp.where(x > 0, x, jnp.exp(jnp.minimum(x, 0.0)) - 1.0)


def _pc(body, out_shape, **kw):
    return pl.pallas_call(body, out_shape=out_shape, interpret=_INTERPRET, **kw)


def _k1_body(e_ref, asrc_ref, adst_ref, we_ref, went_ref, rel_ref, arel_ref,
             w1_ref, oar_ref, tsrc_ref, tdst_ref, ew_ref, re_ref,
             r1p_ref):
    e = e_ref[...]
    nrm = jnp.sqrt(jnp.sum(e * e, axis=1, keepdims=True))
    ent = e / jnp.maximum(nrm, 1e-12)
    dot = functools.partial(jnp.dot, preferred_element_type=F32)
    ue = dot(ent, we_ref[...])
    pad = jnp.zeros((N_NODES, TW - 2 * D1), F32)
    tsrc_ref[...] = jnp.concatenate([dot(ent, asrc_ref[...]), ue, pad], axis=1)
    tdst_ref[...] = jnp.concatenate([dot(ent, adst_ref[...]), ue, pad], axis=1)
    ew_ref[...] = dot(ent, went_ref[...])
    rel = rel_ref[...]
    re_ref[...] = dot(rel, arel_ref[...])
    r1p_ref[...] = dot(dot(rel, w1_ref[...]), oar_ref[...])


def _k1(ent_emb, asrc, adst, we, went, rel, arel, w1, oar):
    outs = (
        jax.ShapeDtypeStruct((N_NODES, TW), F32),
        jax.ShapeDtypeStruct((N_NODES, TW), F32),
        jax.ShapeDtypeStruct((N_NODES, D1), F32),
        jax.ShapeDtypeStruct((N_REL, D1), F32),
        jax.ShapeDtypeStruct((N_REL, D1), F32),
    )
    return _pc(_k1_body, outs)(ent_emb, asrc, adst, we, went, rel, arel, w1,
                               oar)


def _att_edge_body(gs_ref, gd_ref, tf_ref, retab_ref, a2m_ref, hmap_ref,
                   pay_ref):
    em = gs_ref[:, :D1] + gd_ref[:, :D1]
    tf = tf_ref[...]
    oh = jnp.where(
        tf == lax.broadcasted_iota(jnp.int32, (EB, N_REL), 1).astype(F32),
        1.0, 0.0)
    dot = functools.partial(jnp.dot, preferred_element_type=F32)
    em = em + dot(oh, retab_ref[...])
    s = dot(em, a2m_ref[...])
    ee = jnp.exp(-_leaky(s))
    mult = dot(ee, hmap_ref[...])
    nh = ee.shape[1]
    pay_ref[...] = jnp.concatenate(
        [em * mult, ee, jnp.zeros((EB, PW - D1 - nh), F32)], axis=1)


def _att_edge(gs, gd, tf, retab, a2m, hmap):
    nh = a2m.shape[1]
    grid = (NBE,)
    return _pc(
        _att_edge_body,
        jax.ShapeDtypeStruct((N_EDGES, PW), F32),
        grid=grid,
        in_specs=[
            pl.BlockSpec((EB, TW), lambda i: (i, 0)),
            pl.BlockSpec((EB, TW), lambda i: (i, 0)),
            pl.BlockSpec((EB, 1), lambda i: (i, 0)),
            pl.BlockSpec((N_REL, D1), lambda i: (0, 0)),
            pl.BlockSpec((D1, nh), lambda i: (0, 0)),
            pl.BlockSpec((nh, D1), lambda i: (0, 0)),
        ],
        out_specs=pl.BlockSpec((EB, PW), lambda i: (i, 0)),
    )(gs, gd, tf, retab, a2m, hmap)


def _k4_body(acc_ref, oas_ref, oad_ref, xs_ref, xd_ref):
    a = acc_ref[0] + acc_ref[1]
    hp0 = a[:, :NHID] / (a[:, D1:D1 + 1] + 1e-12)
    hp1 = a[:, NHID:D1] / (a[:, D1 + 1:D1 + 2] + 1e-12)
    x = _elu(jnp.concatenate([hp0, hp1], axis=1))
    dot = functools.partial(jnp.dot, preferred_element_type=F32)
    pad = jnp.zeros((NPAD, TW - D1), F32)
    xs_ref[...] = jnp.concatenate([dot(x, oas_ref[...]), pad], axis=1)
    xd_ref[...] = jnp.concatenate([dot(x, oad_ref[...]), pad], axis=1)


def _k4(acc1, oas, oad):
    outs = (jax.ShapeDtypeStruct((NPAD, TW), F32),
            jax.ShapeDtypeStruct((NPAD, TW), F32))
    return _pc(_k4_body, outs)(acc1, oas, oad)


def _k8_body(acc_ref, ew_ref, out_ref):
    a = (acc_ref[0] + acc_ref[1])[:N_NODES]
    hp = a[:, :D1] / (a[:, D1:D1 + 1] + 1e-12)
    mask = jnp.minimum(a[:, 64:65], 1.0)
    oe = ew_ref[...] + mask * _elu(hp)
    nrm = jnp.sqrt(jnp.sum(oe * oe, axis=1, keepdims=True))
    out_ref[...] = oe / jnp.maximum(nrm, 1e-12)


def _k8(acc2, ew):
    return _pc(_k8_body, jax.ShapeDtypeStruct((N_NODES, D1), F32))(acc2, ew)


def _conv_t(gs, gd, i32eye, mp, bp):
    dot = functools.partial(jax.lax.dot_general,
                            preferred_element_type=F32)
    h = gs[:, D1:2 * D1]
    t = gd[:, D1:2 * D1]
    hT = dot(i32eye, h, dimension_numbers=(((1,), (1,)), ((), ())))
    kT = dot(mp, t, dimension_numbers=(((1,), (1,)), ((), ()))) + bp
    rows = []
    for o in range(OUT_CH):
        accum = None
        for w in range(FILT_W):
            term = hT[w:w + CONV_W, :] * kT[w * OUT_CH + o:w * OUT_CH + o + 1, :]
            accum = term if accum is None else accum + term
        rows.append(accum)
    crT = jnp.concatenate(rows, axis=0)
    return hT, kT, crT


def _h1_body(gs_ref, gd_ref, i32_ref, mp_ref, bp_ref, s_ref,
             st768_ref, st32_ref, a768_ref, a32_ref):
    i = pl.program_id(0)

    @pl.when(i == 0)
    def _init():
        a768_ref[...] = jnp.zeros_like(a768_ref)
        a32_ref[...] = jnp.zeros_like(a32_ref)

    hT, kT, crT = _conv_t(gs_ref[...], gd_ref[...], i32_ref[...], mp_ref[...],
                          bp_ref[...])
    dot = functools.partial(jax.lax.dot_general, preferred_element_type=F32)
    smat = s_ref[...]
    crj = dot(smat, crT, dimension_numbers=(((1,), (0,)), ((), ())))
    ksumT = jnp.zeros((OUT_CH, EB), F32)
    for w in range(FILT_W):
        ksumT = ksumT + kT[w * OUT_CH:(w + 1) * OUT_CH, :]
    a1 = jnp.sum(crj, axis=1, keepdims=True)
    a2v = jnp.sum(crT * crT, axis=1, keepdims=True)
    x1 = jnp.sum(crj * ksumT, axis=1, keepdims=True)
    k1v = jnp.sum(ksumT, axis=1, keepdims=True)
    k2v = jnp.sum(ksumT * ksumT, axis=1, keepdims=True)
    hs1 = jnp.sum(hT)
    hs2 = jnp.sum(hT * hT)
    riota = lax.broadcasted_iota(jnp.int32, (OUT_CH, 1), 0)
    hcol = jnp.where(riota == 0, hs1, jnp.where(riota == 1, hs2, 0.0))
    a768_ref[...] += jnp.concatenate(
        [jnp.zeros((NCR, 1), F32), a2v], axis=1)
    a32_ref[...] += jnp.concatenate([a1, x1, k1v, k2v, hcol], axis=1)

    @pl.when(i == NBE - 1)
    def _fin():
        st768_ref[...] = a768_ref[...]
        st32_ref[...] = a32_ref[...]


def _h1(gs, gd, i32eye, mp, bp, smat):
    outs = (jax.ShapeDtypeStruct((NCR, 2), F32),
            jax.ShapeDtypeStruct((OUT_CH, 5), F32))
    return _pc(
        _h1_body, outs,
        grid=(NBE,),
        in_specs=[
            pl.BlockSpec((EB, TW), lambda i: (i, 0)),
            pl.BlockSpec((EB, TW), lambda i: (i, 0)),
            pl.BlockSpec((D1, D1), lambda i: (0, 0)),
            pl.BlockSpec((FILT_W * OUT_CH, D1), lambda i: (0, 0)),
            pl.BlockSpec((FILT_W * OUT_CH, 1), lambda i: (0, 0)),
            pl.BlockSpec((OUT_CH, NCR), lambda i: (0, 0)),
        ],
        out_specs=(pl.BlockSpec((NCR, 2), lambda i: (0, 0)),
                   pl.BlockSpec((OUT_CH, 5), lambda i: (0, 0))),
        scratch_shapes=[pltpu.VMEM((NCR, 2), F32),
                        pltpu.VMEM((OUT_CH, 5), F32)],
    )(gs, gd, i32eye, mp, bp, smat)


def _kh0_body(st768_ref, st32_ref, s_ref, fr_ref, f1s_ref, jr_ref, fcbr_ref,
              bnp_ref, gm_ref):
    dot = functools.partial(jax.lax.dot_general, preferred_element_type=F32)
    smat = s_ref[...]
    a2 = dot(smat, st768_ref[:, 1:2], dimension_numbers=(((1,), (0,)), ((), ())))
    st32 = st32_ref[...]
    a1 = st32[:, 0:1]
    x1 = st32[:, 1:2]
    k1v = st32[:, 2:3]
    k2v = st32[:, 3:4]
    riota = lax.broadcasted_iota(jnp.int32, (OUT_CH, 1), 0)
    hs1 = jnp.sum(jnp.where(riota == 0, st32[:, 4:5], 0.0))
    hs2 = jnp.sum(jnp.where(riota == 1, st32[:, 4:5], 0.0))
    nh = float(N_EDGES * D1)
    mu0 = hs1 / nh
    v0 = hs2 / nh - mu0 * mu0
    bnp = bnp_ref[...]
    g1 = bnp[:, 0:1]
    b1 = bnp[:, 1:2]
    g0 = jnp.sum(jnp.where(riota == 0, bnp[:, 2:3], 0.0))
    b0 = jnp.sum(jnp.where(riota == 0, bnp[:, 3:4], 0.0))
    alpha = g0 / jnp.sqrt(v0 + EPS)
    beta0 = b0 - alpha * mu0
    m1 = float(N_EDGES * CONV_W)
    s1 = alpha * a1 + CONV_W * beta0 * k1v
    s2 = (alpha * alpha * a2 + 2.0 * alpha * beta0 * x1
          + CONV_W * beta0 * beta0 * k2v)
    mu1 = s1 / m1
    v1 = s2 / m1 - mu1 * mu1
    sc = g1 / jnp.sqrt(v1 + EPS)
    c1 = b1 - sc * mu1
    fr_sc = fr_ref[...] * sc
    gshift = jnp.zeros((D1 + 1, D1 * D1), F32)
    colsum = jnp.zeros((D1 + 1, NCR), F32)
    for w in range(FILT_W):
        term = dot(f1s_ref[pl.ds(w * (D1 + 1), D1 + 1), :], fr_sc,
                   dimension_numbers=(((1,), (0,)), ((), ())))
        colsum = colsum + term
        gshift = gshift + jnp.pad(
            term, ((0, 0), (D1 * w, D1 * D1 - NCR - D1 * w)))
    jred = dot(colsum, jr_ref[...],
               dimension_numbers=(((1,), (0,)), ((), ())))
    wjt2 = dot(fr_ref[...], jr_ref[...],
               dimension_numbers=(((1,), (0,)), ((), ())))
    cvrow = dot(c1, wjt2,
                dimension_numbers=(((0,), (0,)), ((), ()))) + fcbr_ref[...]
    citer = lax.broadcasted_iota(jnp.int32, (D1 + 1, 1), 0)
    cvblk = jnp.where(citer == D1, 1.0, 0.0) * cvrow
    gm_ref[...] = jnp.concatenate(
        [alpha * gshift, beta0 * jred + cvblk], axis=1)


def _kh0(st768, st32, smat, fr, f1s, jr, fcbr, bnp):
    return _pc(_kh0_body,
               jax.ShapeDtypeStruct((D1 + 1, D1 * (D1 + 1)), F32))(
        st768, st32, smat, fr, f1s, jr, fcbr, bnp)


def _h2_body(gs_ref, gd_ref, i32_ref, gm2_ref, outT_ref, st2_ref, acc_ref):
    i = pl.program_id(0)

    @pl.when(i == 0)
    def _init():
        acc_ref[...] = jnp.zeros_like(acc_ref)

    dot = functools.partial(jax.lax.dot_general, preferred_element_type=F32)
    h = gs_ref[:, D1:2 * D1]
    t = gd_ref[:, D1:2 * D1]
    eye = i32_ref[...]
    hT = dot(eye, h, dimension_numbers=(((1,), (1,)), ((), ())))
    tT = dot(eye, t, dimension_numbers=(((1,), (1,)), ((), ())))
    ones = jnp.ones((1, EB), F32)
    haT = jnp.concatenate([hT, ones], axis=0)
    taT = jnp.concatenate([tT, ones], axis=0)
    pt = jnp.concatenate(
        [taT * haT[m:m + 1, :] for m in range(D1 + 1)], axis=0)
    outT = dot(gm2_ref[...], pt,
               dimension_numbers=(((0,), (0,)), ((), ())))
    outT_ref[...] = outT
    s1 = jnp.sum(outT, axis=1, keepdims=True)
    s2 = jnp.sum(outT * outT, axis=1, keepdims=True)
    acc_ref[...] += jnp.concatenate([s1, s2], axis=1)

    @pl.when(i == NBE - 1)
    def _fin():
        st2_ref[...] = acc_ref[...]


def _h2(gs, gd, i32eye, gm2):
    outs = (jax.ShapeDtypeStruct((D1, N_EDGES), F32),
            jax.ShapeDtypeStruct((D1, 2), F32))
    return _pc(
        _h2_body, outs,
        grid=(NBE,),
        in_specs=[
            pl.BlockSpec((EB, TW), lambda i: (i, 0)),
            pl.BlockSpec((EB, TW), lambda i: (i, 0)),
            pl.BlockSpec((D1, D1), lambda i: (0, 0)),
            pl.BlockSpec(((D1 + 1) * (D1 + 1), D1), lambda i: (0, 0)),
        ],
        out_specs=(pl.BlockSpec((D1, EB), lambda i: (0, i)),
                   pl.BlockSpec((D1, 2), lambda i: (0, 0))),
        scratch_shapes=[pltpu.VMEM((D1, 2), F32)],
    )(gs, gd, i32eye, gm2)


def _h3_body(outT_ref, st2_ref, tf_ref, bn2_ref, i64_ref, rel_ref,
             racc_ref, cacc_ref):
    i = pl.program_id(0)

    @pl.when(i == 0)
    def _init():
        racc_ref[...] = jnp.zeros_like(racc_ref)
        cacc_ref[...] = jnp.zeros_like(cacc_ref)

    st2 = st2_ref[...]
    mu2 = st2[:, 0:1] / float(N_EDGES)
    v2 = st2[:, 1:2] / float(N_EDGES) - mu2 * mu2
    sc2 = bn2_ref[:, 0:1] / jnp.sqrt(v2 + EPS)
    b2 = bn2_ref[:, 1:2]
    val = jnp.maximum((outT_ref[...] - mu2) * sc2 + b2, 0.0)
    tf = tf_ref[...]
    oh = jnp.where(
        tf == lax.broadcasted_iota(jnp.int32, (EB, N_REL), 1).astype(F32),
        1.0, 0.0)
    dot = functools.partial(jax.lax.dot_general, preferred_element_type=F32)
    racc_ref[...] += dot(val, oh, dimension_numbers=(((1,), (0,)), ((), ())))
    cacc_ref[0:1, :] += jnp.sum(oh, axis=0, keepdims=True)

    @pl.when(i == NBE - 1)
    def _fin():
        cnt = jnp.maximum(cacc_ref[0:1, :], 1.0)
        rm = racc_ref[...] / cnt
        nrm = jnp.sqrt(jnp.sum(rm * rm, axis=0, keepdims=True))
        rn = rm / jnp.maximum(nrm, 1e-12)
        rel_ref[...] = dot(i64_ref[...], rn,
                           dimension_numbers=(((1,), (1,)), ((), ())))


def _h3(outT, st2, tf, bn2, i64eye):
    return _pc(
        _h3_body,
        jax.ShapeDtypeStruct((N_REL, D1), F32),
        grid=(NBE,),
        in_specs=[
            pl.BlockSpec((D1, EB), lambda i: (0, i)),
            pl.BlockSpec((D1, 2), lambda i: (0, 0)),
            pl.BlockSpec((EB, 1), lambda i: (i, 0)),
            pl.BlockSpec((D1, 2), lambda i: (0, 0)),
            pl.BlockSpec((N_REL, N_REL), lambda i: (0, 0)),
        ],
        out_specs=pl.BlockSpec((N_REL, D1), lambda i: (0, 0)),
        scratch_shapes=[pltpu.VMEM((D1, N_REL), F32),
                        pltpu.VMEM((8, N_REL), F32)],
    )(outT, st2, tf, bn2, i64eye)


def _sc_gather(table, idx):
    n, d = table.shape
    e = idx.shape[0]
    nch = e // CH
    base_n = nch // 32
    rem = nch % 32
    mesh = plsc.VectorSubcoreMesh(core_axis_name="c", subcore_axis_name="s")

    @functools.partial(
        pl.kernel, mesh=mesh,
        out_type=jax.ShapeDtypeStruct((e, d), F32),
        scratch_types=[pltpu.VMEM((CH,), jnp.int32),
                       pltpu.VMEM((CH, d), F32),
                       pltpu.SemaphoreType.DMA],
    )
    def k(table_hbm, idx_hbm, out_hbm, idx_v, rows_v, sem):
        wid = lax.axis_index("s") * 2 + lax.axis_index("c")

        def chunk(c):
            off = c * CH
            pltpu.sync_copy(idx_hbm.at[pl.ds(off, CH)], idx_v)
            pltpu.async_copy(table_hbm.at[idx_v], rows_v, sem).wait()
            pltpu.sync_copy(rows_v, out_hbm.at[pl.ds(off, CH)])

        def body(t, carry):
            chunk(t * 32 + wid)
            return carry

        lax.fori_loop(0, base_n, body, 0)
        if rem:
            @pl.when(wid < rem)
            def _tail():
                chunk(base_n * 32 + wid)

    return k(table, idx)


def _sc_gather2(tab_a, idx_a, tab_b, idx_b):
    n, d = tab_a.shape
    e = idx_a.shape[0]
    nch = e // CH
    base_n = nch // 32
    rem = nch % 32
    mesh = plsc.VectorSubcoreMesh(core_axis_name="c", subcore_axis_name="s")

    @functools.partial(
        pl.kernel, mesh=mesh,
        out_type=(jax.ShapeDtypeStruct((e, d), F32),
                  jax.ShapeDtypeStruct((e, d), F32)),
        scratch_types=[pltpu.VMEM((CH,), jnp.int32),
                       pltpu.VMEM((CH, d), F32),
                       pltpu.SemaphoreType.DMA],
    )
    def k(ta_hbm, ia_hbm, tb_hbm, ib_hbm, oa_hbm, ob_hbm, idx_v, rows_v, sem):
        wid = lax.axis_index("s") * 2 + lax.axis_index("c")

        def one(t_hbm, i_hbm, o_hbm, off):
            pltpu.sync_copy(i_hbm.at[pl.ds(off, CH)], idx_v)
            pltpu.async_copy(t_hbm.at[idx_v], rows_v, sem).wait()
            pltpu.sync_copy(rows_v, o_hbm.at[pl.ds(off, CH)])

        def chunk(c):
            off = c * CH
            one(ta_hbm, ia_hbm, oa_hbm, off)
            one(tb_hbm, ib_hbm, ob_hbm, off)

        def body(t, carry):
            chunk(t * 32 + wid)
            return carry

        lax.fori_loop(0, base_n, body, 0)
        if rem:
            @pl.when(wid < rem)
            def _tail():
                chunk(base_n * 32 + wid)

    return k(tab_a, idx_a, tab_b, idx_b)


def _sc_scatter_add(vals, idx, n):
    e, d = vals.shape
    nch = e // CH
    base_n = nch // 32
    rem = nch % 32
    rows_t = n // 16
    mesh = plsc.VectorSubcoreMesh(core_axis_name="c", subcore_axis_name="s")
    zeros = jnp.zeros((n, d), F32)

    @functools.partial(
        pl.kernel, mesh=mesh,
        out_type=jax.ShapeDtypeStruct((2, n, d), F32),
        scratch_types=[pltpu.VMEM((CH,), jnp.int32),
                       pltpu.VMEM((CH, d), F32),
                       pltpu.VMEM_SHARED((n, d), F32)],
    )
    def k(vals_hbm, idx_hbm, zero_hbm, out_hbm, idx_v, rows_v, acc_sh):
        cid = lax.axis_index("c")
        sid = lax.axis_index("s")
        wid = sid * 2 + cid
        pltpu.sync_copy(zero_hbm.at[pl.ds(sid * rows_t, rows_t)],
                        acc_sh.at[pl.ds(sid * rows_t, rows_t)])
        plsc.subcore_barrier()

        def chunk(c):
            off = c * CH
            pltpu.sync_copy(idx_hbm.at[pl.ds(off, CH)], idx_v)
            pltpu.sync_copy(vals_hbm.at[pl.ds(off, CH)], rows_v)
            pltpu.sync_copy(rows_v, acc_sh.at[idx_v], add=True)

        def body(t, carry):
            chunk(t * 32 + wid)
            return carry

        lax.fori_loop(0, base_n, body, 0)
        if rem:
            @pl.when(wid < rem)
            def _tail():
                chunk(base_n * 32 + wid)

        plsc.subcore_barrier()
        pltpu.sync_copy(acc_sh.at[pl.ds(sid * rows_t, rows_t)],
                        out_hbm.at[cid].at[pl.ds(sid * rows_t, rows_t)])

    return k(vals, idx, zeros)


def _sc_scatter_add2(vals, idx, vals2, idx2, n):
    e, d = vals.shape
    nch = e // CH
    base_n = nch // 32
    rem = nch % 32
    e2 = vals2.shape[0]
    nch2 = e2 // CH
    base2 = nch2 // 32
    rem2 = nch2 % 32
    rows_t = n // 16
    mesh = plsc.VectorSubcoreMesh(core_axis_name="c", subcore_axis_name="s")
    zeros = jnp.zeros((n, d), F32)

    @functools.partial(
        pl.kernel, mesh=mesh,
        out_type=jax.ShapeDtypeStruct((2, n, d), F32),
        scratch_types=[pltpu.VMEM((CH,), jnp.int32),
                       pltpu.VMEM((CH, d), F32),
                       pltpu.VMEM_SHARED((n, d), F32)],
    )
    def k(vals_hbm, idx_hbm, v2_hbm, i2_hbm, zero_hbm, out_hbm,
          idx_v, rows_v, acc_sh):
        cid = lax.axis_index("c")
        sid = lax.axis_index("s")
        wid = sid * 2 + cid
        pltpu.sync_copy(zero_hbm.at[pl.ds(sid * rows_t, rows_t)],
                        acc_sh.at[pl.ds(sid * rows_t, rows_t)])
        plsc.subcore_barrier()

        def chunk(v_hbm, i_hbm, c):
            off = c * CH
            pltpu.sync_copy(i_hbm.at[pl.ds(off, CH)], idx_v)
            pltpu.sync_copy(v_hbm.at[pl.ds(off, CH)], rows_v)
            pltpu.sync_copy(rows_v, acc_sh.at[idx_v], add=True)

        def body(t, carry):
            chunk(vals_hbm, idx_hbm, t * 32 + wid)
            return carry

        lax.fori_loop(0, base_n, body, 0)
        if rem:
            @pl.when(wid < rem)
            def _tail():
                chunk(vals_hbm, idx_hbm, base_n * 32 + wid)

        def body2(t, carry):
            chunk(v2_hbm, i2_hbm, t * 32 + wid)
            return carry

        lax.fori_loop(0, base2, body2, 0)
        if rem2:
            @pl.when(wid < rem2)
            def _tail2():
                chunk(v2_hbm, i2_hbm, base2 * 32 + wid)

        plsc.subcore_barrier()
        pltpu.sync_copy(acc_sh.at[pl.ds(sid * rows_t, rows_t)],
                        out_hbm.at[cid].at[pl.ds(sid * rows_t, rows_t)])

    return k(vals, idx, vals2, idx2, zeros)


def _emu_scatter(vals, idx, n):
    out = jnp.zeros((2, n, vals.shape[1]), F32)
    return out.at[0].set(jax.ops.segment_sum(vals, idx, num_segments=n))


def kernel(edge_list, edge_type, batch_inputs, params):
    p = params
    src = edge_list[0].astype(jnp.int32)
    dst = edge_list[1].astype(jnp.int32)
    tf = edge_type.astype(F32).reshape(N_EDGES, 1)

    aa = p['att_a']
    asrc = jnp.concatenate([aa[0, :, :IN_DIM], aa[1, :, :IN_DIM]], 0).T
    adst = jnp.concatenate([aa[0, :, IN_DIM:2 * IN_DIM],
                            aa[1, :, IN_DIM:2 * IN_DIM]], 0).T
    arel = jnp.concatenate([aa[0, :, 2 * IN_DIM:], aa[1, :, 2 * IN_DIM:]], 0).T
    a2 = p['att_a2']
    a2m = jnp.zeros((D1, NHEADS), F32)
    a2m = a2m.at[:NHID, 0].set(a2[0, 0]).at[NHID:, 1].set(a2[1, 0])
    hmap = jnp.zeros((NHEADS, D1), F32)
    hmap = hmap.at[0, :NHID].set(1.0).at[1, NHID:].set(1.0)
    oa = p['out_a']
    oas, oad, oar = oa[:, :D1].T, oa[:, D1:2 * D1].T, oa[:, 2 * D1:].T
    oa2m = p['out_a2'].T
    h1map = jnp.ones((1, D1), F32)
    mp = p['fc1_w'].T.reshape(OUT_CH, FILT_W, D1).transpose(1, 0, 2) \
        .reshape(FILT_W * OUT_CH, D1)
    bp = p['fc1_b'].reshape(OUT_CH, FILT_W).T.reshape(FILT_W * OUT_CH, 1)
    i32eye = jnp.eye(D1, dtype=F32)
    i64eye = jnp.eye(N_REL, dtype=F32)
    smat = jnp.kron(jnp.eye(OUT_CH, dtype=F32), jnp.ones((1, CONV_W), F32))
    fr = p['fc_w'].reshape(OUT_CH, CONV_W * D1)
    f1aug = jnp.concatenate([p['fc1_w'], p['fc1_b'].reshape(1, -1)], axis=0)
    f1s = f1aug.reshape(D1 + 1, OUT_CH, FILT_W).transpose(2, 0, 1).reshape(
        FILT_W * (D1 + 1), OUT_CH)
    jr = jnp.tile(jnp.eye(D1, dtype=F32), (CONV_W, 1))
    fcbr = p['fc_b'].reshape(1, D1)
    bnp = jnp.concatenate([
        p['bn1_g'].reshape(OUT_CH, 1), p['bn1_b'].reshape(OUT_CH, 1),
        jnp.full((OUT_CH, 1), p['bn0_g'][0]),
        jnp.full((OUT_CH, 1), p['bn0_b'][0])], axis=1)
    bn2 = jnp.concatenate([p['bn2_g'].reshape(D1, 1),
                           p['bn2_b'].reshape(D1, 1)], axis=1)

    tsrc, tdst, ew, re, r1p = _k1(
        p['entity_embeddings'], asrc, adst, p['W_E'], p['W_entities'],
        p['relation_embeddings'], arel, p['W_1'], oar)

    gs, gd = _sc_gather2(tsrc, src, tdst, dst)

    pay1 = _att_edge(gs, gd, tf, re, a2m, hmap)
    acc1 = _sc_scatter_add(pay1, src, NPAD)
    xs, xd = _k4(acc1, oas, oad)

    gs2, gd2 = _sc_gather2(xs, src, xd, dst)
    pay2 = _att_edge(gs2, gd2, tf, r1p, oa2m, h1map)
    mones = jnp.zeros((BATCH, PW), F32).at[:, 64].set(1.0)
    tgt = batch_inputs[:, 2].astype(jnp.int32)
    acc2 = _sc_scatter_add2(pay2, src, mones, tgt, NPAD)

    out_entity = _k8(acc2, ew)

    st768, st32 = _h1(gs, gd, i32eye, mp, bp, smat)
    gm = _kh0(st768, st32, smat, fr, f1s, jr, fcbr, bnp)
    gm2 = gm.reshape(D1 + 1, D1 + 1, D1).transpose(1, 0, 2).reshape(
        (D1 + 1) * (D1 + 1), D1)
    outT, st2 = _h2(gs, gd, i32eye, gm2)
    out_relation = _h3(outT, st2, tf, bn2, i64eye)

    return (out_entity, out_relation)


# speedup vs baseline: 4.1109x; 1.0642x over previous
import functools

import jax
import jax.numpy as jnp
import numpy as np
from jax import lax
from jax.experimental import pallas as pl
from jax.experimental.pallas import tpu as pltpu
from jax.experimental.pallas import tpu_sc as plsc

N_NODES = 10000
N_REL = 64
IN_DIM = 128
REL_DIM = 64
NHID = 16
NHEADS = 2
D1 = NHID * NHEADS
N_EDGES = 160000
BATCH = 8192
ALPHA = 0.2
EPS = 1e-5
FILT_W = 9
OUT_CH = 32
CONV_W = D1 - FILT_W + 1
NCR = OUT_CH * CONV_W

EB = 1280
NBE = N_EDGES // EB
PW = 128
TW = 128
NPAD = 10240
CH = 128
F32 = jnp.float32

_INTERPRET = False


def _leaky(x):
    return jnp.where(x > 0, x, ALPHA * x)


def _elu(x):
    return jnp.where(x > 0, x, jnp.exp(jnp.minimum(x, 0.0)) - 1.0)


def _pc(body, out_shape, **kw):
    return pl.pallas_call(body, out_shape=out_shape, interpret=_INTERPRET, **kw)


def _k1_body(e_ref, asrc_ref, adst_ref, we_ref, went_ref, rel_ref, arel_ref,
             w1_ref, oar_ref, tsrc_ref, tdst_ref, ew_ref, re_ref,
             r1p_ref):
    e = e_ref[...]
    nrm = jnp.sqrt(jnp.sum(e * e, axis=1, keepdims=True))
    ent = e / jnp.maximum(nrm, 1e-12)
    dot = functools.partial(jnp.dot, preferred_element_type=F32)
    ue = dot(ent, we_ref[...])
    pad = jnp.zeros((N_NODES, TW - 2 * D1), F32)
    tsrc_ref[...] = jnp.concatenate([dot(ent, asrc_ref[...]), ue, pad], axis=1)
    tdst_ref[...] = jnp.concatenate([dot(ent, adst_ref[...]), ue, pad], axis=1)
    ew_ref[...] = dot(ent, went_ref[...])
    rel = rel_ref[...]
    re_ref[...] = dot(rel, arel_ref[...])
    r1p_ref[...] = dot(dot(rel, w1_ref[...]), oar_ref[...])


def _k1(ent_emb, asrc, adst, we, went, rel, arel, w1, oar):
    outs = (
        jax.ShapeDtypeStruct((N_NODES, TW), F32),
        jax.ShapeDtypeStruct((N_NODES, TW), F32),
        jax.ShapeDtypeStruct((N_NODES, D1), F32),
        jax.ShapeDtypeStruct((N_REL, D1), F32),
        jax.ShapeDtypeStruct((N_REL, D1), F32),
    )
    return _pc(_k1_body, outs)(ent_emb, asrc, adst, we, went, rel, arel, w1,
                               oar)


def _att_edge_body(gs_ref, gd_ref, tf_ref, retab_ref, a2m_ref, hmap_ref,
                   pay_ref):
    em = gs_ref[:, :D1] + gd_ref[:, :D1]
    tf = tf_ref[...]
    oh = jnp.where(
        tf == lax.broadcasted_iota(jnp.int32, (EB, N_REL), 1).astype(F32),
        1.0, 0.0)
    dot = functools.partial(jnp.dot, preferred_element_type=F32)
    em = em + dot(oh, retab_ref[...])
    s = dot(em, a2m_ref[...])
    ee = jnp.exp(-_leaky(s))
    mult = dot(ee, hmap_ref[...])
    nh = ee.shape[1]
    pay_ref[...] = jnp.concatenate(
        [em * mult, ee, jnp.zeros((EB, PW - D1 - nh), F32)], axis=1)


def _att_edge(gs, gd, tf, retab, a2m, hmap):
    nh = a2m.shape[1]
    grid = (NBE,)
    return _pc(
        _att_edge_body,
        jax.ShapeDtypeStruct((N_EDGES, PW), F32),
        grid=grid,
        in_specs=[
            pl.BlockSpec((EB, TW), lambda i: (i, 0)),
            pl.BlockSpec((EB, TW), lambda i: (i, 0)),
            pl.BlockSpec((EB, 1), lambda i: (i, 0)),
            pl.BlockSpec((N_REL, D1), lambda i: (0, 0)),
            pl.BlockSpec((D1, nh), lambda i: (0, 0)),
            pl.BlockSpec((nh, D1), lambda i: (0, 0)),
        ],
        out_specs=pl.BlockSpec((EB, PW), lambda i: (i, 0)),
    )(gs, gd, tf, retab, a2m, hmap)


def _k4_body(acc_ref, oas_ref, oad_ref, xs_ref, xd_ref):
    a = acc_ref[0] + acc_ref[1]
    hp0 = a[:, :NHID] / (a[:, D1:D1 + 1] + 1e-12)
    hp1 = a[:, NHID:D1] / (a[:, D1 + 1:D1 + 2] + 1e-12)
    x = _elu(jnp.concatenate([hp0, hp1], axis=1))
    dot = functools.partial(jnp.dot, preferred_element_type=F32)
    pad = jnp.zeros((NPAD, TW - D1), F32)
    xs_ref[...] = jnp.concatenate([dot(x, oas_ref[...]), pad], axis=1)
    xd_ref[...] = jnp.concatenate([dot(x, oad_ref[...]), pad], axis=1)


def _k4(acc1, oas, oad):
    outs = (jax.ShapeDtypeStruct((NPAD, TW), F32),
            jax.ShapeDtypeStruct((NPAD, TW), F32))
    return _pc(_k4_body, outs)(acc1, oas, oad)


def _k8_body(acc_ref, ew_ref, out_ref):
    a = (acc_ref[0] + acc_ref[1])[:N_NODES]
    hp = a[:, :D1] / (a[:, D1:D1 + 1] + 1e-12)
    mask = jnp.minimum(a[:, 64:65], 1.0)
    oe = ew_ref[...] + mask * _elu(hp)
    nrm = jnp.sqrt(jnp.sum(oe * oe, axis=1, keepdims=True))
    out_ref[...] = oe / jnp.maximum(nrm, 1e-12)


def _k8(acc2, ew):
    return _pc(_k8_body, jax.ShapeDtypeStruct((N_NODES, D1), F32))(acc2, ew)


def _conv_t(gs, gd, i32eye, mp, bp):
    dot = functools.partial(jax.lax.dot_general,
                            preferred_element_type=F32)
    h = gs[:, D1:2 * D1]
    t = gd[:, D1:2 * D1]
    hT = dot(i32eye, h, dimension_numbers=(((1,), (1,)), ((), ())))
    kT = dot(mp, t, dimension_numbers=(((1,), (1,)), ((), ()))) + bp
    rows = []
    for o in range(OUT_CH):
        accum = None
        for w in range(FILT_W):
            term = hT[w:w + CONV_W, :] * kT[w * OUT_CH + o:w * OUT_CH + o + 1, :]
            accum = term if accum is None else accum + term
        rows.append(accum)
    crT = jnp.concatenate(rows, axis=0)
    return hT, kT, crT


def _ah1_body(gs_ref, gd_ref, tf_ref, retab_ref, a2m_ref, hmap_ref,
              i32_ref, mp_ref, bp_ref, s_ref,
              pay_ref, st768_ref, st32_ref, a768_ref, a32_ref):
    i = pl.program_id(0)

    @pl.when(i == 0)
    def _init():
        a768_ref[...] = jnp.zeros_like(a768_ref)
        a32_ref[...] = jnp.zeros_like(a32_ref)

    gs = gs_ref[...]
    gd = gd_ref[...]

    em = gs[:, :D1] + gd[:, :D1]
    tf = tf_ref[...]
    oh = jnp.where(
        tf == lax.broadcasted_iota(jnp.int32, (EB, N_REL), 1).astype(F32),
        1.0, 0.0)
    dotp = functools.partial(jnp.dot, preferred_element_type=F32)
    em = em + dotp(oh, retab_ref[...])
    s = dotp(em, a2m_ref[...])
    ee = jnp.exp(-_leaky(s))
    mult = dotp(ee, hmap_ref[...])
    nh = ee.shape[1]
    pay_ref[...] = jnp.concatenate(
        [em * mult, ee, jnp.zeros((EB, PW - D1 - nh), F32)], axis=1)

    hT, kT, crT = _conv_t(gs, gd, i32_ref[...], mp_ref[...], bp_ref[...])
    dot = functools.partial(jax.lax.dot_general, preferred_element_type=F32)
    smat = s_ref[...]
    crj = dot(smat, crT, dimension_numbers=(((1,), (0,)), ((), ())))
    ksumT = jnp.zeros((OUT_CH, EB), F32)
    for w in range(FILT_W):
        ksumT = ksumT + kT[w * OUT_CH:(w + 1) * OUT_CH, :]
    a1 = jnp.sum(crj, axis=1, keepdims=True)
    a2v = jnp.sum(crT * crT, axis=1, keepdims=True)
    x1 = jnp.sum(crj * ksumT, axis=1, keepdims=True)
    k1v = jnp.sum(ksumT, axis=1, keepdims=True)
    k2v = jnp.sum(ksumT * ksumT, axis=1, keepdims=True)
    hs1 = jnp.sum(hT)
    hs2 = jnp.sum(hT * hT)
    riota = lax.broadcasted_iota(jnp.int32, (OUT_CH, 1), 0)
    hcol = jnp.where(riota == 0, hs1, jnp.where(riota == 1, hs2, 0.0))
    a768_ref[...] += jnp.concatenate(
        [jnp.zeros((NCR, 1), F32), a2v], axis=1)
    a32_ref[...] += jnp.concatenate([a1, x1, k1v, k2v, hcol], axis=1)

    @pl.when(i == NBE - 1)
    def _fin():
        st768_ref[...] = a768_ref[...]
        st32_ref[...] = a32_ref[...]


def _ah1(gs, gd, tf, retab, a2m, hmap, i32eye, mp, bp, smat):
    nh = a2m.shape[1]
    outs = (jax.ShapeDtypeStruct((N_EDGES, PW), F32),
            jax.ShapeDtypeStruct((NCR, 2), F32),
            jax.ShapeDtypeStruct((OUT_CH, 5), F32))
    return _pc(
        _ah1_body, outs,
        grid=(NBE,),
        in_specs=[
            pl.BlockSpec((EB, TW), lambda i: (i, 0)),
            pl.BlockSpec((EB, TW), lambda i: (i, 0)),
            pl.BlockSpec((EB, 1), lambda i: (i, 0)),
            pl.BlockSpec((N_REL, D1), lambda i: (0, 0)),
            pl.BlockSpec((D1, nh), lambda i: (0, 0)),
            pl.BlockSpec((nh, D1), lambda i: (0, 0)),
            pl.BlockSpec((D1, D1), lambda i: (0, 0)),
            pl.BlockSpec((FILT_W * OUT_CH, D1), lambda i: (0, 0)),
            pl.BlockSpec((FILT_W * OUT_CH, 1), lambda i: (0, 0)),
            pl.BlockSpec((OUT_CH, NCR), lambda i: (0, 0)),
        ],
        out_specs=(pl.BlockSpec((EB, PW), lambda i: (i, 0)),
                   pl.BlockSpec((NCR, 2), lambda i: (0, 0)),
                   pl.BlockSpec((OUT_CH, 5), lambda i: (0, 0))),
        scratch_shapes=[pltpu.VMEM((NCR, 2), F32),
                        pltpu.VMEM((OUT_CH, 5), F32)],
    )(gs, gd, tf, retab, a2m, hmap, i32eye, mp, bp, smat)


def _kh0_body(st768_ref, st32_ref, s_ref, fr_ref, f1s_ref, jr_ref, fcbr_ref,
              bnp_ref, gm_ref):
    dot = functools.partial(jax.lax.dot_general, preferred_element_type=F32)
    smat = s_ref[...]
    a2 = dot(smat, st768_ref[:, 1:2], dimension_numbers=(((1,), (0,)), ((), ())))
    st32 = st32_ref[...]
    a1 = st32[:, 0:1]
    x1 = st32[:, 1:2]
    k1v = st32[:, 2:3]
    k2v = st32[:, 3:4]
    riota = lax.broadcasted_iota(jnp.int32, (OUT_CH, 1), 0)
    hs1 = jnp.sum(jnp.where(riota == 0, st32[:, 4:5], 0.0))
    hs2 = jnp.sum(jnp.where(riota == 1, st32[:, 4:5], 0.0))
    nh = float(N_EDGES * D1)
    mu0 = hs1 / nh
    v0 = hs2 / nh - mu0 * mu0
    bnp = bnp_ref[...]
    g1 = bnp[:, 0:1]
    b1 = bnp[:, 1:2]
    g0 = jnp.sum(jnp.where(riota == 0, bnp[:, 2:3], 0.0))
    b0 = jnp.sum(jnp.where(riota == 0, bnp[:, 3:4], 0.0))
    alpha = g0 / jnp.sqrt(v0 + EPS)
    beta0 = b0 - alpha * mu0
    m1 = float(N_EDGES * CONV_W)
    s1 = alpha * a1 + CONV_W * beta0 * k1v
    s2 = (alpha * alpha * a2 + 2.0 * alpha * beta0 * x1
          + CONV_W * beta0 * beta0 * k2v)
    mu1 = s1 / m1
    v1 = s2 / m1 - mu1 * mu1
    sc = g1 / jnp.sqrt(v1 + EPS)
    c1 = b1 - sc * mu1
    fr_sc = fr_ref[...] * sc
    gshift = jnp.zeros((D1 + 1, D1 * D1), F32)
    colsum = jnp.zeros((D1 + 1, NCR), F32)
    for w in range(FILT_W):
        term = dot(f1s_ref[pl.ds(w * (D1 + 1), D1 + 1), :], fr_sc,
                   dimension_numbers=(((1,), (0,)), ((), ())))
        colsum = colsum + term
        gshift = gshift + jnp.pad(
            term, ((0, 0), (D1 * w, D1 * D1 - NCR - D1 * w)))
    jred = dot(colsum, jr_ref[...],
               dimension_numbers=(((1,), (0,)), ((), ())))
    wjt2 = dot(fr_ref[...], jr_ref[...],
               dimension_numbers=(((1,), (0,)), ((), ())))
    cvrow = dot(c1, wjt2,
                dimension_numbers=(((0,), (0,)), ((), ()))) + fcbr_ref[...]
    citer = lax.broadcasted_iota(jnp.int32, (D1 + 1, 1), 0)
    cvblk = jnp.where(citer == D1, 1.0, 0.0) * cvrow
    gm_ref[...] = jnp.concatenate(
        [alpha * gshift, beta0 * jred + cvblk], axis=1)


def _kh0(st768, st32, smat, fr, f1s, jr, fcbr, bnp):
    return _pc(_kh0_body,
               jax.ShapeDtypeStruct((D1 + 1, D1 * (D1 + 1)), F32))(
        st768, st32, smat, fr, f1s, jr, fcbr, bnp)


def _h2_body(gs_ref, gd_ref, i32_ref, gm2_ref, outT_ref, st2_ref, acc_ref):
    i = pl.program_id(0)

    @pl.when(i == 0)
    def _init():
        acc_ref[...] = jnp.zeros_like(acc_ref)

    dot = functools.partial(jax.lax.dot_general, preferred_element_type=F32)
    h = gs_ref[:, D1:2 * D1]
    t = gd_ref[:, D1:2 * D1]
    eye = i32_ref[...]
    hT = dot(eye, h, dimension_numbers=(((1,), (1,)), ((), ())))
    tT = dot(eye, t, dimension_numbers=(((1,), (1,)), ((), ())))
    ones = jnp.ones((1, EB), F32)
    haT = jnp.concatenate([hT, ones], axis=0)
    taT = jnp.concatenate([tT, ones], axis=0)
    pt = jnp.concatenate(
        [taT * haT[m:m + 1, :] for m in range(D1 + 1)], axis=0)
    outT = dot(gm2_ref[...], pt,
               dimension_numbers=(((0,), (0,)), ((), ())))
    outT_ref[...] = outT
    s1 = jnp.sum(outT, axis=1, keepdims=True)
    s2 = jnp.sum(outT * outT, axis=1, keepdims=True)
    acc_ref[...] += jnp.concatenate([s1, s2], axis=1)

    @pl.when(i == NBE - 1)
    def _fin():
        st2_ref[...] = acc_ref[...]


def _h2(gs, gd, i32eye, gm2):
    outs = (jax.ShapeDtypeStruct((D1, N_EDGES), F32),
            jax.ShapeDtypeStruct((D1, 2), F32))
    return _pc(
        _h2_body, outs,
        grid=(NBE,),
        in_specs=[
            pl.BlockSpec((EB, TW), lambda i: (i, 0)),
            pl.BlockSpec((EB, TW), lambda i: (i, 0)),
            pl.BlockSpec((D1, D1), lambda i: (0, 0)),
            pl.BlockSpec(((D1 + 1) * (D1 + 1), D1), lambda i: (0, 0)),
        ],
        out_specs=(pl.BlockSpec((D1, EB), lambda i: (0, i)),
                   pl.BlockSpec((D1, 2), lambda i: (0, 0))),
        scratch_shapes=[pltpu.VMEM((D1, 2), F32)],
    )(gs, gd, i32eye, gm2)


def _h3_body(outT_ref, st2_ref, tf_ref, bn2_ref, i64_ref, rel_ref,
             racc_ref, cacc_ref):
    i = pl.program_id(0)

    @pl.when(i == 0)
    def _init():
        racc_ref[...] = jnp.zeros_like(racc_ref)
        cacc_ref[...] = jnp.zeros_like(cacc_ref)

    st2 = st2_ref[...]
    mu2 = st2[:, 0:1] / float(N_EDGES)
    v2 = st2[:, 1:2] / float(N_EDGES) - mu2 * mu2
    sc2 = bn2_ref[:, 0:1] / jnp.sqrt(v2 + EPS)
    b2 = bn2_ref[:, 1:2]
    val = jnp.maximum((outT_ref[...] - mu2) * sc2 + b2, 0.0)
    tf = tf_ref[...]
    oh = jnp.where(
        tf == lax.broadcasted_iota(jnp.int32, (EB, N_REL), 1).astype(F32),
        1.0, 0.0)
    dot = functools.partial(jax.lax.dot_general, preferred_element_type=F32)
    racc_ref[...] += dot(val, oh, dimension_numbers=(((1,), (0,)), ((), ())))
    cacc_ref[0:1, :] += jnp.sum(oh, axis=0, keepdims=True)

    @pl.when(i == NBE - 1)
    def _fin():
        cnt = jnp.maximum(cacc_ref[0:1, :], 1.0)
        rm = racc_ref[...] / cnt
        nrm = jnp.sqrt(jnp.sum(rm * rm, axis=0, keepdims=True))
        rn = rm / jnp.maximum(nrm, 1e-12)
        rel_ref[...] = dot(i64_ref[...], rn,
                           dimension_numbers=(((1,), (1,)), ((), ())))


def _h3(outT, st2, tf, bn2, i64eye):
    return _pc(
        _h3_body,
        jax.ShapeDtypeStruct((N_REL, D1), F32),
        grid=(NBE,),
        in_specs=[
            pl.BlockSpec((D1, EB), lambda i: (0, i)),
            pl.BlockSpec((D1, 2), lambda i: (0, 0)),
            pl.BlockSpec((EB, 1), lambda i: (i, 0)),
            pl.BlockSpec((D1, 2), lambda i: (0, 0)),
            pl.BlockSpec((N_REL, N_REL), lambda i: (0, 0)),
        ],
        out_specs=pl.BlockSpec((N_REL, D1), lambda i: (0, 0)),
        scratch_shapes=[pltpu.VMEM((D1, N_REL), F32),
                        pltpu.VMEM((8, N_REL), F32)],
    )(outT, st2, tf, bn2, i64eye)


def _sc_gather(table, idx):
    n, d = table.shape
    e = idx.shape[0]
    nch = e // CH
    base_n = nch // 32
    rem = nch % 32
    mesh = plsc.VectorSubcoreMesh(core_axis_name="c", subcore_axis_name="s")

    @functools.partial(
        pl.kernel, mesh=mesh,
        out_type=jax.ShapeDtypeStruct((e, d), F32),
        scratch_types=[pltpu.VMEM((CH,), jnp.int32),
                       pltpu.VMEM((CH, d), F32),
                       pltpu.SemaphoreType.DMA],
    )
    def k(table_hbm, idx_hbm, out_hbm, idx_v, rows_v, sem):
        wid = lax.axis_index("s") * 2 + lax.axis_index("c")

        def chunk(c):
            off = c * CH
            pltpu.sync_copy(idx_hbm.at[pl.ds(off, CH)], idx_v)
            pltpu.async_copy(table_hbm.at[idx_v], rows_v, sem).wait()
            pltpu.sync_copy(rows_v, out_hbm.at[pl.ds(off, CH)])

        def body(t, carry):
            chunk(t * 32 + wid)
            return carry

        lax.fori_loop(0, base_n, body, 0)
        if rem:
            @pl.when(wid < rem)
            def _tail():
                chunk(base_n * 32 + wid)

    return k(table, idx)


def _sc_gather2(tab_a, idx_a, tab_b, idx_b):
    n, d = tab_a.shape
    e = idx_a.shape[0]
    nch = e // CH
    base_n = nch // 32
    rem = nch % 32
    mesh = plsc.VectorSubcoreMesh(core_axis_name="c", subcore_axis_name="s")

    @functools.partial(
        pl.kernel, mesh=mesh,
        out_type=(jax.ShapeDtypeStruct((e, d), F32),
                  jax.ShapeDtypeStruct((e, d), F32)),
        scratch_types=[pltpu.VMEM((CH,), jnp.int32),
                       pltpu.VMEM((CH, d), F32),
                       pltpu.SemaphoreType.DMA],
    )
    def k(ta_hbm, ia_hbm, tb_hbm, ib_hbm, oa_hbm, ob_hbm, idx_v, rows_v, sem):
        wid = lax.axis_index("s") * 2 + lax.axis_index("c")

        def one(t_hbm, i_hbm, o_hbm, off):
            pltpu.sync_copy(i_hbm.at[pl.ds(off, CH)], idx_v)
            pltpu.async_copy(t_hbm.at[idx_v], rows_v, sem).wait()
            pltpu.sync_copy(rows_v, o_hbm.at[pl.ds(off, CH)])

        def chunk(c):
            off = c * CH
            one(ta_hbm, ia_hbm, oa_hbm, off)
            one(tb_hbm, ib_hbm, ob_hbm, off)

        def body(t, carry):
            chunk(t * 32 + wid)
            return carry

        lax.fori_loop(0, base_n, body, 0)
        if rem:
            @pl.when(wid < rem)
            def _tail():
                chunk(base_n * 32 + wid)

    return k(tab_a, idx_a, tab_b, idx_b)


def _sc_scatter_add(vals, idx, n):
    e, d = vals.shape
    nch = e // CH
    base_n = nch // 32
    rem = nch % 32
    rows_t = n // 16
    mesh = plsc.VectorSubcoreMesh(core_axis_name="c", subcore_axis_name="s")
    zeros = jnp.zeros((n, d), F32)

    @functools.partial(
        pl.kernel, mesh=mesh,
        out_type=jax.ShapeDtypeStruct((2, n, d), F32),
        scratch_types=[pltpu.VMEM((CH,), jnp.int32),
                       pltpu.VMEM((CH, d), F32),
                       pltpu.VMEM_SHARED((n, d), F32)],
    )
    def k(vals_hbm, idx_hbm, zero_hbm, out_hbm, idx_v, rows_v, acc_sh):
        cid = lax.axis_index("c")
        sid = lax.axis_index("s")
        wid = sid * 2 + cid
        pltpu.sync_copy(zero_hbm.at[pl.ds(sid * rows_t, rows_t)],
                        acc_sh.at[pl.ds(sid * rows_t, rows_t)])
        plsc.subcore_barrier()

        def chunk(c):
            off = c * CH
            pltpu.sync_copy(idx_hbm.at[pl.ds(off, CH)], idx_v)
            pltpu.sync_copy(vals_hbm.at[pl.ds(off, CH)], rows_v)
            pltpu.sync_copy(rows_v, acc_sh.at[idx_v], add=True)

        def body(t, carry):
            chunk(t * 32 + wid)
            return carry

        lax.fori_loop(0, base_n, body, 0)
        if rem:
            @pl.when(wid < rem)
            def _tail():
                chunk(base_n * 32 + wid)

        plsc.subcore_barrier()
        pltpu.sync_copy(acc_sh.at[pl.ds(sid * rows_t, rows_t)],
                        out_hbm.at[cid].at[pl.ds(sid * rows_t, rows_t)])

    return k(vals, idx, zeros)


def _sc_scatter_add2(vals, idx, vals2, idx2, n):
    e, d = vals.shape
    nch = e // CH
    base_n = nch // 32
    rem = nch % 32
    e2 = vals2.shape[0]
    nch2 = e2 // CH
    base2 = nch2 // 32
    rem2 = nch2 % 32
    rows_t = n // 16
    mesh = plsc.VectorSubcoreMesh(core_axis_name="c", subcore_axis_name="s")
    zeros = jnp.zeros((n, d), F32)

    @functools.partial(
        pl.kernel, mesh=mesh,
        out_type=jax.ShapeDtypeStruct((2, n, d), F32),
        scratch_types=[pltpu.VMEM((CH,), jnp.int32),
                       pltpu.VMEM((CH, d), F32),
                       pltpu.VMEM_SHARED((n, d), F32)],
    )
    def k(vals_hbm, idx_hbm, v2_hbm, i2_hbm, zero_hbm, out_hbm,
          idx_v, rows_v, acc_sh):
        cid = lax.axis_index("c")
        sid = lax.axis_index("s")
        wid = sid * 2 + cid
        pltpu.sync_copy(zero_hbm.at[pl.ds(sid * rows_t, rows_t)],
                        acc_sh.at[pl.ds(sid * rows_t, rows_t)])
        plsc.subcore_barrier()

        def chunk(v_hbm, i_hbm, c):
            off = c * CH
            pltpu.sync_copy(i_hbm.at[pl.ds(off, CH)], idx_v)
            pltpu.sync_copy(v_hbm.at[pl.ds(off, CH)], rows_v)
            pltpu.sync_copy(rows_v, acc_sh.at[idx_v], add=True)

        def body(t, carry):
            chunk(vals_hbm, idx_hbm, t * 32 + wid)
            return carry

        lax.fori_loop(0, base_n, body, 0)
        if rem:
            @pl.when(wid < rem)
            def _tail():
                chunk(vals_hbm, idx_hbm, base_n * 32 + wid)

        def body2(t, carry):
            chunk(v2_hbm, i2_hbm, t * 32 + wid)
            return carry

        lax.fori_loop(0, base2, body2, 0)
        if rem2:
            @pl.when(wid < rem2)
            def _tail2():
                chunk(v2_hbm, i2_hbm, base2 * 32 + wid)

        plsc.subcore_barrier()
        pltpu.sync_copy(acc_sh.at[pl.ds(sid * rows_t, rows_t)],
                        out_hbm.at[cid].at[pl.ds(sid * rows_t, rows_t)])

    return k(vals, idx, vals2, idx2, zeros)


def _emu_scatter(vals, idx, n):
    out = jnp.zeros((2, n, vals.shape[1]), F32)
    return out.at[0].set(jax.ops.segment_sum(vals, idx, num_segments=n))


def kernel(edge_list, edge_type, batch_inputs, params):
    p = params
    src = edge_list[0].astype(jnp.int32)
    dst = edge_list[1].astype(jnp.int32)
    tf = edge_type.astype(F32).reshape(N_EDGES, 1)

    aa = p['att_a']
    asrc = jnp.concatenate([aa[0, :, :IN_DIM], aa[1, :, :IN_DIM]], 0).T
    adst = jnp.concatenate([aa[0, :, IN_DIM:2 * IN_DIM],
                            aa[1, :, IN_DIM:2 * IN_DIM]], 0).T
    arel = jnp.concatenate([aa[0, :, 2 * IN_DIM:], aa[1, :, 2 * IN_DIM:]], 0).T
    a2 = p['att_a2']
    a2m = jnp.zeros((D1, NHEADS), F32)
    a2m = a2m.at[:NHID, 0].set(a2[0, 0]).at[NHID:, 1].set(a2[1, 0])
    hmap = jnp.zeros((NHEADS, D1), F32)
    hmap = hmap.at[0, :NHID].set(1.0).at[1, NHID:].set(1.0)
    oa = p['out_a']
    oas, oad, oar = oa[:, :D1].T, oa[:, D1:2 * D1].T, oa[:, 2 * D1:].T
    oa2m = p['out_a2'].T
    h1map = jnp.ones((1, D1), F32)
    mp = p['fc1_w'].T.reshape(OUT_CH, FILT_W, D1).transpose(1, 0, 2) \
        .reshape(FILT_W * OUT_CH, D1)
    bp = p['fc1_b'].reshape(OUT_CH, FILT_W).T.reshape(FILT_W * OUT_CH, 1)
    i32eye = jnp.eye(D1, dtype=F32)
    i64eye = jnp.eye(N_REL, dtype=F32)
    smat = jnp.kron(jnp.eye(OUT_CH, dtype=F32), jnp.ones((1, CONV_W), F32))
    fr = p['fc_w'].reshape(OUT_CH, CONV_W * D1)
    f1aug = jnp.concatenate([p['fc1_w'], p['fc1_b'].reshape(1, -1)], axis=0)
    f1s = f1aug.reshape(D1 + 1, OUT_CH, FILT_W).transpose(2, 0, 1).reshape(
        FILT_W * (D1 + 1), OUT_CH)
    jr = jnp.tile(jnp.eye(D1, dtype=F32), (CONV_W, 1))
    fcbr = p['fc_b'].reshape(1, D1)
    bnp = jnp.concatenate([
        p['bn1_g'].reshape(OUT_CH, 1), p['bn1_b'].reshape(OUT_CH, 1),
        jnp.full((OUT_CH, 1), p['bn0_g'][0]),
        jnp.full((OUT_CH, 1), p['bn0_b'][0])], axis=1)
    bn2 = jnp.concatenate([p['bn2_g'].reshape(D1, 1),
                           p['bn2_b'].reshape(D1, 1)], axis=1)

    tsrc, tdst, ew, re, r1p = _k1(
        p['entity_embeddings'], asrc, adst, p['W_E'], p['W_entities'],
        p['relation_embeddings'], arel, p['W_1'], oar)

    gs, gd = _sc_gather2(tsrc, src, tdst, dst)

    pay1, st768, st32 = _ah1(gs, gd, tf, re, a2m, hmap, i32eye, mp, bp, smat)
    acc1 = _sc_scatter_add(pay1, src, NPAD)
    xs, xd = _k4(acc1, oas, oad)

    gs2, gd2 = _sc_gather2(xs, src, xd, dst)
    pay2 = _att_edge(gs2, gd2, tf, r1p, oa2m, h1map)
    mones = jnp.zeros((BATCH, PW), F32).at[:, 64].set(1.0)
    tgt = batch_inputs[:, 2].astype(jnp.int32)
    acc2 = _sc_scatter_add2(pay2, src, mones, tgt, NPAD)

    out_entity = _k8(acc2, ew)

    gm = _kh0(st768, st32, smat, fr, f1s, jr, fcbr, bnp)
    gm2 = gm.reshape(D1 + 1, D1 + 1, D1).transpose(1, 0, 2).reshape(
        (D1 + 1) * (D1 + 1), D1)
    outT, st2 = _h2(gs, gd, i32eye, gm2)
    out_relation = _h3(outT, st2, tf, bn2, i64eye)

    return (out_entity, out_relation)


# speedup vs baseline: 4.1684x; 1.0140x over previous
import functools

import jax
import jax.numpy as jnp
import numpy as np
from jax import lax
from jax.experimental import pallas as pl
from jax.experimental.pallas import tpu as pltpu
from jax.experimental.pallas import tpu_sc as plsc

N_NODES = 10000
N_REL = 64
IN_DIM = 128
REL_DIM = 64
NHID = 16
NHEADS = 2
D1 = NHID * NHEADS
N_EDGES = 160000
BATCH = 8192
ALPHA = 0.2
EPS = 1e-5
FILT_W = 9
OUT_CH = 32
CONV_W = D1 - FILT_W + 1
NCR = OUT_CH * CONV_W

EB = 1280
NBE = N_EDGES // EB
PW = 128
TW = 128
NPAD = 10240
CH = 128
F32 = jnp.float32

_INTERPRET = False


def _leaky(x):
    return jnp.where(x > 0, x, ALPHA * x)


def _elu(x):
    return jnp.where(x > 0, x, jnp.exp(jnp.minimum(x, 0.0)) - 1.0)


def _pc(body, out_shape, **kw):
    return pl.pallas_call(body, out_shape=out_shape, interpret=_INTERPRET, **kw)


def _k1_body(e_ref, asrc_ref, adst_ref, we_ref, went_ref, rel_ref, arel_ref,
             w1_ref, oar_ref, tsrc_ref, tdst_ref, ew_ref, re_ref,
             r1p_ref):
    e = e_ref[...]
    nrm = jnp.sqrt(jnp.sum(e * e, axis=1, keepdims=True))
    ent = e / jnp.maximum(nrm, 1e-12)
    dot = functools.partial(jnp.dot, preferred_element_type=F32)
    ue = dot(ent, we_ref[...])
    pad = jnp.zeros((N_NODES, TW - 2 * D1), F32)
    tsrc_ref[...] = jnp.concatenate([dot(ent, asrc_ref[...]), ue, pad], axis=1)
    tdst_ref[...] = jnp.concatenate([dot(ent, adst_ref[...]), ue, pad], axis=1)
    ew_ref[...] = dot(ent, went_ref[...])
    rel = rel_ref[...]
    re_ref[...] = dot(rel, arel_ref[...])
    r1p_ref[...] = dot(dot(rel, w1_ref[...]), oar_ref[...])


def _k1(ent_emb, asrc, adst, we, went, rel, arel, w1, oar):
    outs = (
        jax.ShapeDtypeStruct((N_NODES, TW), F32),
        jax.ShapeDtypeStruct((N_NODES, TW), F32),
        jax.ShapeDtypeStruct((N_NODES, D1), F32),
        jax.ShapeDtypeStruct((N_REL, D1), F32),
        jax.ShapeDtypeStruct((N_REL, D1), F32),
    )
    return _pc(_k1_body, outs)(ent_emb, asrc, adst, we, went, rel, arel, w1,
                               oar)


def _att_edge_body(gs_ref, gd_ref, tf_ref, retab_ref, a2m_ref, hmap_ref,
                   pay_ref):
    em = gs_ref[:, :D1] + gd_ref[:, :D1]
    tf = tf_ref[...]
    oh = jnp.where(
        tf == lax.broadcasted_iota(jnp.int32, (EB, N_REL), 1).astype(F32),
        1.0, 0.0)
    dot = functools.partial(jnp.dot, preferred_element_type=F32)
    em = em + dot(oh, retab_ref[...])
    s = dot(em, a2m_ref[...])
    ee = jnp.exp(-_leaky(s))
    mult = dot(ee, hmap_ref[...])
    nh = ee.shape[1]
    pay_ref[...] = jnp.concatenate(
        [em * mult, ee, jnp.zeros((EB, PW - D1 - nh), F32)], axis=1)


def _att_edge(gs, gd, tf, retab, a2m, hmap):
    nh = a2m.shape[1]
    grid = (NBE,)
    return _pc(
        _att_edge_body,
        jax.ShapeDtypeStruct((N_EDGES, PW), F32),
        grid=grid,
        in_specs=[
            pl.BlockSpec((EB, TW), lambda i: (i, 0)),
            pl.BlockSpec((EB, TW), lambda i: (i, 0)),
            pl.BlockSpec((EB, 1), lambda i: (i, 0)),
            pl.BlockSpec((N_REL, D1), lambda i: (0, 0)),
            pl.BlockSpec((D1, nh), lambda i: (0, 0)),
            pl.BlockSpec((nh, D1), lambda i: (0, 0)),
        ],
        out_specs=pl.BlockSpec((EB, PW), lambda i: (i, 0)),
    )(gs, gd, tf, retab, a2m, hmap)


def _k4_body(acc_ref, oas_ref, oad_ref, xs_ref, xd_ref):
    a = acc_ref[0] + acc_ref[1]
    hp0 = a[:, :NHID] / (a[:, D1:D1 + 1] + 1e-12)
    hp1 = a[:, NHID:D1] / (a[:, D1 + 1:D1 + 2] + 1e-12)
    x = _elu(jnp.concatenate([hp0, hp1], axis=1))
    dot = functools.partial(jnp.dot, preferred_element_type=F32)
    pad = jnp.zeros((NPAD, TW - D1), F32)
    xs_ref[...] = jnp.concatenate([dot(x, oas_ref[...]), pad], axis=1)
    xd_ref[...] = jnp.concatenate([dot(x, oad_ref[...]), pad], axis=1)


def _k4(acc1, oas, oad):
    outs = (jax.ShapeDtypeStruct((NPAD, TW), F32),
            jax.ShapeDtypeStruct((NPAD, TW), F32))
    return _pc(_k4_body, outs)(acc1, oas, oad)


def _k8_body(acc_ref, ew_ref, out_ref):
    a = (acc_ref[0] + acc_ref[1])[:N_NODES]
    hp = a[:, :D1] / (a[:, D1:D1 + 1] + 1e-12)
    mask = jnp.minimum(a[:, 64:65], 1.0)
    oe = ew_ref[...] + mask * _elu(hp)
    nrm = jnp.sqrt(jnp.sum(oe * oe, axis=1, keepdims=True))
    out_ref[...] = oe / jnp.maximum(nrm, 1e-12)


def _k8(acc2, ew):
    return _pc(_k8_body, jax.ShapeDtypeStruct((N_NODES, D1), F32))(acc2, ew)


def _conv_t(gs, gd, i32eye, mp, bp):
    dot = functools.partial(jax.lax.dot_general,
                            preferred_element_type=F32)
    h = gs[:, D1:2 * D1]
    t = gd[:, D1:2 * D1]
    hT = dot(i32eye, h, dimension_numbers=(((1,), (1,)), ((), ())))
    kT = dot(mp, t, dimension_numbers=(((1,), (1,)), ((), ()))) + bp
    rows = []
    for o in range(OUT_CH):
        accum = None
        for w in range(FILT_W):
            term = hT[w:w + CONV_W, :] * kT[w * OUT_CH + o:w * OUT_CH + o + 1, :]
            accum = term if accum is None else accum + term
        rows.append(accum)
    crT = jnp.concatenate(rows, axis=0)
    return hT, kT, crT


def _ah1_body(gs_ref, gd_ref, tf_ref, retab_ref, a2m_ref, hmap_ref,
              i32_ref, mp_ref, bp_ref, s_ref,
              pay_ref, ht_ref, st768_ref, st32_ref, a768_ref, a32_ref):
    i = pl.program_id(0)

    @pl.when(i == 0)
    def _init():
        a768_ref[...] = jnp.zeros_like(a768_ref)
        a32_ref[...] = jnp.zeros_like(a32_ref)

    gs = gs_ref[...]
    gd = gd_ref[...]

    em = gs[:, :D1] + gd[:, :D1]
    tf = tf_ref[...]
    oh = jnp.where(
        tf == lax.broadcasted_iota(jnp.int32, (EB, N_REL), 1).astype(F32),
        1.0, 0.0)
    dotp = functools.partial(jnp.dot, preferred_element_type=F32)
    em = em + dotp(oh, retab_ref[...])
    s = dotp(em, a2m_ref[...])
    ee = jnp.exp(-_leaky(s))
    mult = dotp(ee, hmap_ref[...])
    nh = ee.shape[1]
    pay_ref[...] = jnp.concatenate(
        [em * mult, ee, jnp.zeros((EB, PW - D1 - nh), F32)], axis=1)

    ht_ref[...] = jnp.concatenate(
        [gs[:, D1:2 * D1], gd[:, D1:2 * D1]], axis=1)

    hT, kT, crT = _conv_t(gs, gd, i32_ref[...], mp_ref[...], bp_ref[...])
    dot = functools.partial(jax.lax.dot_general, preferred_element_type=F32)
    smat = s_ref[...]
    crj = dot(smat, crT, dimension_numbers=(((1,), (0,)), ((), ())))
    ksumT = jnp.zeros((OUT_CH, EB), F32)
    for w in range(FILT_W):
        ksumT = ksumT + kT[w * OUT_CH:(w + 1) * OUT_CH, :]
    a1 = jnp.sum(crj, axis=1, keepdims=True)
    a2v = jnp.sum(crT * crT, axis=1, keepdims=True)
    x1 = jnp.sum(crj * ksumT, axis=1, keepdims=True)
    k1v = jnp.sum(ksumT, axis=1, keepdims=True)
    k2v = jnp.sum(ksumT * ksumT, axis=1, keepdims=True)
    hs1 = jnp.sum(hT)
    hs2 = jnp.sum(hT * hT)
    riota = lax.broadcasted_iota(jnp.int32, (OUT_CH, 1), 0)
    hcol = jnp.where(riota == 0, hs1, jnp.where(riota == 1, hs2, 0.0))
    a768_ref[...] += jnp.concatenate(
        [jnp.zeros((NCR, 1), F32), a2v], axis=1)
    a32_ref[...] += jnp.concatenate([a1, x1, k1v, k2v, hcol], axis=1)

    @pl.when(i == NBE - 1)
    def _fin():
        st768_ref[...] = a768_ref[...]
        st32_ref[...] = a32_ref[...]


def _ah1(gs, gd, tf, retab, a2m, hmap, i32eye, mp, bp, smat):
    nh = a2m.shape[1]
    outs = (jax.ShapeDtypeStruct((N_EDGES, PW), F32),
            jax.ShapeDtypeStruct((N_EDGES, 2 * D1), F32),
            jax.ShapeDtypeStruct((NCR, 2), F32),
            jax.ShapeDtypeStruct((OUT_CH, 5), F32))
    return _pc(
        _ah1_body, outs,
        grid=(NBE,),
        in_specs=[
            pl.BlockSpec((EB, TW), lambda i: (i, 0)),
            pl.BlockSpec((EB, TW), lambda i: (i, 0)),
            pl.BlockSpec((EB, 1), lambda i: (i, 0)),
            pl.BlockSpec((N_REL, D1), lambda i: (0, 0)),
            pl.BlockSpec((D1, nh), lambda i: (0, 0)),
            pl.BlockSpec((nh, D1), lambda i: (0, 0)),
            pl.BlockSpec((D1, D1), lambda i: (0, 0)),
            pl.BlockSpec((FILT_W * OUT_CH, D1), lambda i: (0, 0)),
            pl.BlockSpec((FILT_W * OUT_CH, 1), lambda i: (0, 0)),
            pl.BlockSpec((OUT_CH, NCR), lambda i: (0, 0)),
        ],
        out_specs=(pl.BlockSpec((EB, PW), lambda i: (i, 0)),
                   pl.BlockSpec((EB, 2 * D1), lambda i: (i, 0)),
                   pl.BlockSpec((NCR, 2), lambda i: (0, 0)),
                   pl.BlockSpec((OUT_CH, 5), lambda i: (0, 0))),
        scratch_shapes=[pltpu.VMEM((NCR, 2), F32),
                        pltpu.VMEM((OUT_CH, 5), F32)],
    )(gs, gd, tf, retab, a2m, hmap, i32eye, mp, bp, smat)


def _kh0_body(st768_ref, st32_ref, s_ref, fr_ref, f1s_ref, jr_ref, fcbr_ref,
              bnp_ref, gm_ref):
    dot = functools.partial(jax.lax.dot_general, preferred_element_type=F32)
    smat = s_ref[...]
    a2 = dot(smat, st768_ref[:, 1:2], dimension_numbers=(((1,), (0,)), ((), ())))
    st32 = st32_ref[...]
    a1 = st32[:, 0:1]
    x1 = st32[:, 1:2]
    k1v = st32[:, 2:3]
    k2v = st32[:, 3:4]
    riota = lax.broadcasted_iota(jnp.int32, (OUT_CH, 1), 0)
    hs1 = jnp.sum(jnp.where(riota == 0, st32[:, 4:5], 0.0))
    hs2 = jnp.sum(jnp.where(riota == 1, st32[:, 4:5], 0.0))
    nh = float(N_EDGES * D1)
    mu0 = hs1 / nh
    v0 = hs2 / nh - mu0 * mu0
    bnp = bnp_ref[...]
    g1 = bnp[:, 0:1]
    b1 = bnp[:, 1:2]
    g0 = jnp.sum(jnp.where(riota == 0, bnp[:, 2:3], 0.0))
    b0 = jnp.sum(jnp.where(riota == 0, bnp[:, 3:4], 0.0))
    alpha = g0 / jnp.sqrt(v0 + EPS)
    beta0 = b0 - alpha * mu0
    m1 = float(N_EDGES * CONV_W)
    s1 = alpha * a1 + CONV_W * beta0 * k1v
    s2 = (alpha * alpha * a2 + 2.0 * alpha * beta0 * x1
          + CONV_W * beta0 * beta0 * k2v)
    mu1 = s1 / m1
    v1 = s2 / m1 - mu1 * mu1
    sc = g1 / jnp.sqrt(v1 + EPS)
    c1 = b1 - sc * mu1
    fr_sc = fr_ref[...] * sc
    gshift = jnp.zeros((D1 + 1, D1 * D1), F32)
    colsum = jnp.zeros((D1 + 1, NCR), F32)
    for w in range(FILT_W):
        term = dot(f1s_ref[pl.ds(w * (D1 + 1), D1 + 1), :], fr_sc,
                   dimension_numbers=(((1,), (0,)), ((), ())))
        colsum = colsum + term
        gshift = gshift + jnp.pad(
            term, ((0, 0), (D1 * w, D1 * D1 - NCR - D1 * w)))
    jred = dot(colsum, jr_ref[...],
               dimension_numbers=(((1,), (0,)), ((), ())))
    wjt2 = dot(fr_ref[...], jr_ref[...],
               dimension_numbers=(((1,), (0,)), ((), ())))
    cvrow = dot(c1, wjt2,
                dimension_numbers=(((0,), (0,)), ((), ()))) + fcbr_ref[...]
    citer = lax.broadcasted_iota(jnp.int32, (D1 + 1, 1), 0)
    cvblk = jnp.where(citer == D1, 1.0, 0.0) * cvrow
    gm_ref[...] = jnp.concatenate(
        [alpha * gshift, beta0 * jred + cvblk], axis=1)


def _kh0(st768, st32, smat, fr, f1s, jr, fcbr, bnp):
    return _pc(_kh0_body,
               jax.ShapeDtypeStruct((D1 + 1, D1 * (D1 + 1)), F32))(
        st768, st32, smat, fr, f1s, jr, fcbr, bnp)


def _h23_body(ht_ref, i32_ref, gm2_ref, tf_ref, bn2_ref, i64_ref, rel_ref,
              outT_s, st2_s, racc_ref, cacc_ref):
    i = pl.program_id(0)
    dot = functools.partial(jax.lax.dot_general, preferred_element_type=F32)

    @pl.when(i == 0)
    def _init():
        st2_s[...] = jnp.zeros_like(st2_s)
        racc_ref[...] = jnp.zeros_like(racc_ref)
        cacc_ref[...] = jnp.zeros_like(cacc_ref)

    @pl.when(i < NBE)
    def _apply():
        ht = ht_ref[...]
        eye = i32_ref[...]
        hT = dot(eye, ht[:, :D1], dimension_numbers=(((1,), (1,)), ((), ())))
        tT = dot(eye, ht[:, D1:], dimension_numbers=(((1,), (1,)), ((), ())))
        ones = jnp.ones((1, EB), F32)
        haT = jnp.concatenate([hT, ones], axis=0)
        taT = jnp.concatenate([tT, ones], axis=0)
        pt = jnp.concatenate(
            [taT * haT[m:m + 1, :] for m in range(D1 + 1)], axis=0)
        outT = dot(gm2_ref[...], pt,
                   dimension_numbers=(((0,), (0,)), ((), ())))
        outT_s[:, pl.ds(i * EB, EB)] = outT
        s1 = jnp.sum(outT, axis=1, keepdims=True)
        s2 = jnp.sum(outT * outT, axis=1, keepdims=True)
        st2_s[...] += jnp.concatenate([s1, s2], axis=1)

    @pl.when(i >= NBE)
    def _reduce():
        j = i - NBE
        st2 = st2_s[...]
        mu2 = st2[:, 0:1] / float(N_EDGES)
        v2 = st2[:, 1:2] / float(N_EDGES) - mu2 * mu2
        sc2 = bn2_ref[:, 0:1] / jnp.sqrt(v2 + EPS)
        b2 = bn2_ref[:, 1:2]
        outT = outT_s[:, pl.ds(j * EB, EB)]
        val = jnp.maximum((outT - mu2) * sc2 + b2, 0.0)
        tf = tf_ref[...]
        oh = jnp.where(
            tf == lax.broadcasted_iota(jnp.int32, (EB, N_REL), 1).astype(F32),
            1.0, 0.0)
        racc_ref[...] += dot(val, oh,
                             dimension_numbers=(((1,), (0,)), ((), ())))
        cacc_ref[0:1, :] += jnp.sum(oh, axis=0, keepdims=True)

    @pl.when(i == 2 * NBE - 1)
    def _fin():
        cnt = jnp.maximum(cacc_ref[0:1, :], 1.0)
        rm = racc_ref[...] / cnt
        nrm = jnp.sqrt(jnp.sum(rm * rm, axis=0, keepdims=True))
        rn = rm / jnp.maximum(nrm, 1e-12)
        rel_ref[...] = dot(i64_ref[...], rn,
                           dimension_numbers=(((1,), (1,)), ((), ())))


def _h23(ht, i32eye, gm2, tf, bn2, i64eye):
    return _pc(
        _h23_body,
        jax.ShapeDtypeStruct((N_REL, D1), F32),
        grid=(2 * NBE,),
        in_specs=[
            pl.BlockSpec((EB, 2 * D1), lambda i: (jnp.minimum(i, NBE - 1), 0)),
            pl.BlockSpec((D1, D1), lambda i: (0, 0)),
            pl.BlockSpec(((D1 + 1) * (D1 + 1), D1), lambda i: (0, 0)),
            pl.BlockSpec((EB, 1), lambda i: (jnp.maximum(i - NBE, 0), 0)),
            pl.BlockSpec((D1, 2), lambda i: (0, 0)),
            pl.BlockSpec((N_REL, N_REL), lambda i: (0, 0)),
        ],
        out_specs=pl.BlockSpec((N_REL, D1), lambda i: (0, 0)),
        scratch_shapes=[pltpu.VMEM((D1, N_EDGES), F32),
                        pltpu.VMEM((D1, 2), F32),
                        pltpu.VMEM((D1, N_REL), F32),
                        pltpu.VMEM((8, N_REL), F32)],
    )(ht, i32eye, gm2, tf, bn2, i64eye)


def _sc_gather(table, idx):
    n, d = table.shape
    e = idx.shape[0]
    nch = e // CH
    base_n = nch // 32
    rem = nch % 32
    mesh = plsc.VectorSubcoreMesh(core_axis_name="c", subcore_axis_name="s")

    @functools.partial(
        pl.kernel, mesh=mesh,
        out_type=jax.ShapeDtypeStruct((e, d), F32),
        scratch_types=[pltpu.VMEM((CH,), jnp.int32),
                       pltpu.VMEM((CH, d), F32),
                       pltpu.SemaphoreType.DMA],
    )
    def k(table_hbm, idx_hbm, out_hbm, idx_v, rows_v, sem):
        wid = lax.axis_index("s") * 2 + lax.axis_index("c")

        def chunk(c):
            off = c * CH
            pltpu.sync_copy(idx_hbm.at[pl.ds(off, CH)], idx_v)
            pltpu.async_copy(table_hbm.at[idx_v], rows_v, sem).wait()
            pltpu.sync_copy(rows_v, out_hbm.at[pl.ds(off, CH)])

        def body(t, carry):
            chunk(t * 32 + wid)
            return carry

        lax.fori_loop(0, base_n, body, 0)
        if rem:
            @pl.when(wid < rem)
            def _tail():
                chunk(base_n * 32 + wid)

    return k(table, idx)


def _sc_gather2(tab_a, idx_a, tab_b, idx_b):
    n, d = tab_a.shape
    e = idx_a.shape[0]
    nch = e // CH
    base_n = nch // 32
    rem = nch % 32
    mesh = plsc.VectorSubcoreMesh(core_axis_name="c", subcore_axis_name="s")

    @functools.partial(
        pl.kernel, mesh=mesh,
        out_type=(jax.ShapeDtypeStruct((e, d), F32),
                  jax.ShapeDtypeStruct((e, d), F32)),
        scratch_types=[pltpu.VMEM((CH,), jnp.int32),
                       pltpu.VMEM((CH, d), F32),
                       pltpu.SemaphoreType.DMA],
    )
    def k(ta_hbm, ia_hbm, tb_hbm, ib_hbm, oa_hbm, ob_hbm, idx_v, rows_v, sem):
        wid = lax.axis_index("s") * 2 + lax.axis_index("c")

        def one(t_hbm, i_hbm, o_hbm, off):
            pltpu.sync_copy(i_hbm.at[pl.ds(off, CH)], idx_v)
            pltpu.async_copy(t_hbm.at[idx_v], rows_v, sem).wait()
            pltpu.sync_copy(rows_v, o_hbm.at[pl.ds(off, CH)])

        def chunk(c):
            off = c * CH
            one(ta_hbm, ia_hbm, oa_hbm, off)
            one(tb_hbm, ib_hbm, ob_hbm, off)

        def body(t, carry):
            chunk(t * 32 + wid)
            return carry

        lax.fori_loop(0, base_n, body, 0)
        if rem:
            @pl.when(wid < rem)
            def _tail():
                chunk(base_n * 32 + wid)

    return k(tab_a, idx_a, tab_b, idx_b)


def _sc_scatter_add(vals, idx, n):
    e, d = vals.shape
    nch = e // CH
    base_n = nch // 32
    rem = nch % 32
    rows_t = n // 16
    mesh = plsc.VectorSubcoreMesh(core_axis_name="c", subcore_axis_name="s")
    zeros = jnp.zeros((n, d), F32)

    @functools.partial(
        pl.kernel, mesh=mesh,
        out_type=jax.ShapeDtypeStruct((2, n, d), F32),
        scratch_types=[pltpu.VMEM((CH,), jnp.int32),
                       pltpu.VMEM((CH, d), F32),
                       pltpu.VMEM_SHARED((n, d), F32)],
    )
    def k(vals_hbm, idx_hbm, zero_hbm, out_hbm, idx_v, rows_v, acc_sh):
        cid = lax.axis_index("c")
        sid = lax.axis_index("s")
        wid = sid * 2 + cid
        pltpu.sync_copy(zero_hbm.at[pl.ds(sid * rows_t, rows_t)],
                        acc_sh.at[pl.ds(sid * rows_t, rows_t)])
        plsc.subcore_barrier()

        def chunk(c):
            off = c * CH
            pltpu.sync_copy(idx_hbm.at[pl.ds(off, CH)], idx_v)
            pltpu.sync_copy(vals_hbm.at[pl.ds(off, CH)], rows_v)
            pltpu.sync_copy(rows_v, acc_sh.at[idx_v], add=True)

        def body(t, carry):
            chunk(t * 32 + wid)
            return carry

        lax.fori_loop(0, base_n, body, 0)
        if rem:
            @pl.when(wid < rem)
            def _tail():
                chunk(base_n * 32 + wid)

        plsc.subcore_barrier()
        pltpu.sync_copy(acc_sh.at[pl.ds(sid * rows_t, rows_t)],
                        out_hbm.at[cid].at[pl.ds(sid * rows_t, rows_t)])

    return k(vals, idx, zeros)


def _sc_scatter_add2(vals, idx, vals2, idx2, n):
    e, d = vals.shape
    nch = e // CH
    base_n = nch // 32
    rem = nch % 32
    e2 = vals2.shape[0]
    nch2 = e2 // CH
    base2 = nch2 // 32
    rem2 = nch2 % 32
    rows_t = n // 16
    mesh = plsc.VectorSubcoreMesh(core_axis_name="c", subcore_axis_name="s")
    zeros = jnp.zeros((n, d), F32)

    @functools.partial(
        pl.kernel, mesh=mesh,
        out_type=jax.ShapeDtypeStruct((2, n, d), F32),
        scratch_types=[pltpu.VMEM((CH,), jnp.int32),
                       pltpu.VMEM((CH, d), F32),
                       pltpu.VMEM_SHARED((n, d), F32)],
    )
    def k(vals_hbm, idx_hbm, v2_hbm, i2_hbm, zero_hbm, out_hbm,
          idx_v, rows_v, acc_sh):
        cid = lax.axis_index("c")
        sid = lax.axis_index("s")
        wid = sid * 2 + cid
        pltpu.sync_copy(zero_hbm.at[pl.ds(sid * rows_t, rows_t)],
                        acc_sh.at[pl.ds(sid * rows_t, rows_t)])
        plsc.subcore_barrier()

        def chunk(v_hbm, i_hbm, c):
            off = c * CH
            pltpu.sync_copy(i_hbm.at[pl.ds(off, CH)], idx_v)
            pltpu.sync_copy(v_hbm.at[pl.ds(off, CH)], rows_v)
            pltpu.sync_copy(rows_v, acc_sh.at[idx_v], add=True)

        def body(t, carry):
            chunk(vals_hbm, idx_hbm, t * 32 + wid)
            return carry

        lax.fori_loop(0, base_n, body, 0)
        if rem:
            @pl.when(wid < rem)
            def _tail():
                chunk(vals_hbm, idx_hbm, base_n * 32 + wid)

        def body2(t, carry):
            chunk(v2_hbm, i2_hbm, t * 32 + wid)
            return carry

        lax.fori_loop(0, base2, body2, 0)
        if rem2:
            @pl.when(wid < rem2)
            def _tail2():
                chunk(v2_hbm, i2_hbm, base2 * 32 + wid)

        plsc.subcore_barrier()
        pltpu.sync_copy(acc_sh.at[pl.ds(sid * rows_t, rows_t)],
                        out_hbm.at[cid].at[pl.ds(sid * rows_t, rows_t)])

    return k(vals, idx, vals2, idx2, zeros)


def _emu_scatter(vals, idx, n):
    out = jnp.zeros((2, n, vals.shape[1]), F32)
    return out.at[0].set(jax.ops.segment_sum(vals, idx, num_segments=n))


def kernel(edge_list, edge_type, batch_inputs, params):
    p = params
    src = edge_list[0].astype(jnp.int32)
    dst = edge_list[1].astype(jnp.int32)
    tf = edge_type.astype(F32).reshape(N_EDGES, 1)

    aa = p['att_a']
    asrc = jnp.concatenate([aa[0, :, :IN_DIM], aa[1, :, :IN_DIM]], 0).T
    adst = jnp.concatenate([aa[0, :, IN_DIM:2 * IN_DIM],
                            aa[1, :, IN_DIM:2 * IN_DIM]], 0).T
    arel = jnp.concatenate([aa[0, :, 2 * IN_DIM:], aa[1, :, 2 * IN_DIM:]], 0).T
    a2 = p['att_a2']
    a2m = jnp.zeros((D1, NHEADS), F32)
    a2m = a2m.at[:NHID, 0].set(a2[0, 0]).at[NHID:, 1].set(a2[1, 0])
    hmap = jnp.zeros((NHEADS, D1), F32)
    hmap = hmap.at[0, :NHID].set(1.0).at[1, NHID:].set(1.0)
    oa = p['out_a']
    oas, oad, oar = oa[:, :D1].T, oa[:, D1:2 * D1].T, oa[:, 2 * D1:].T
    oa2m = p['out_a2'].T
    h1map = jnp.ones((1, D1), F32)
    mp = p['fc1_w'].T.reshape(OUT_CH, FILT_W, D1).transpose(1, 0, 2) \
        .reshape(FILT_W * OUT_CH, D1)
    bp = p['fc1_b'].reshape(OUT_CH, FILT_W).T.reshape(FILT_W * OUT_CH, 1)
    i32eye = jnp.eye(D1, dtype=F32)
    i64eye = jnp.eye(N_REL, dtype=F32)
    smat = jnp.kron(jnp.eye(OUT_CH, dtype=F32), jnp.ones((1, CONV_W), F32))
    fr = p['fc_w'].reshape(OUT_CH, CONV_W * D1)
    f1aug = jnp.concatenate([p['fc1_w'], p['fc1_b'].reshape(1, -1)], axis=0)
    f1s = f1aug.reshape(D1 + 1, OUT_CH, FILT_W).transpose(2, 0, 1).reshape(
        FILT_W * (D1 + 1), OUT_CH)
    jr = jnp.tile(jnp.eye(D1, dtype=F32), (CONV_W, 1))
    fcbr = p['fc_b'].reshape(1, D1)
    bnp = jnp.concatenate([
        p['bn1_g'].reshape(OUT_CH, 1), p['bn1_b'].reshape(OUT_CH, 1),
        jnp.full((OUT_CH, 1), p['bn0_g'][0]),
        jnp.full((OUT_CH, 1), p['bn0_b'][0])], axis=1)
    bn2 = jnp.concatenate([p['bn2_g'].reshape(D1, 1),
                           p['bn2_b'].reshape(D1, 1)], axis=1)

    tsrc, tdst, ew, re, r1p = _k1(
        p['entity_embeddings'], asrc, adst, p['W_E'], p['W_entities'],
        p['relation_embeddings'], arel, p['W_1'], oar)

    gs, gd = _sc_gather2(tsrc, src, tdst, dst)

    pay1, ht, st768, st32 = _ah1(gs, gd, tf, re, a2m, hmap, i32eye, mp, bp,
                                 smat)
    acc1 = _sc_scatter_add(pay1, src, NPAD)
    xs, xd = _k4(acc1, oas, oad)

    gs2, gd2 = _sc_gather2(xs, src, xd, dst)
    pay2 = _att_edge(gs2, gd2, tf, r1p, oa2m, h1map)
    mones = jnp.zeros((BATCH, PW), F32).at[:, 64].set(1.0)
    tgt = batch_inputs[:, 2].astype(jnp.int32)
    acc2 = _sc_scatter_add2(pay2, src, mones, tgt, NPAD)

    out_entity = _k8(acc2, ew)

    gm = _kh0(st768, st32, smat, fr, f1s, jr, fcbr, bnp)
    gm2 = gm.reshape(D1 + 1, D1 + 1, D1).transpose(1, 0, 2).reshape(
        (D1 + 1) * (D1 + 1), D1)
    out_relation = _h23(ht, i32eye, gm2, tf, bn2, i64eye)

    return (out_entity, out_relation)
